# Initial kernel scaffold; baseline (speedup 1.0000x reference)
import jax
import jax.numpy as jnp
from jax.experimental import pallas as pl


def kernel(x0, x1, edge_attr0, edge_attr1, edge_index0, edge_index1, graph_ids0, graph_ids1, W_nemb, b_nemb, W_eemb, b_eemb, W_proj, b_proj, W_msg1, b_msg1, W_msg2, b_msg2, W_upd, b_upd, W_nout, b_nout, w_atom, b_atom, w_beta, b_beta, W_m1, b_m1, ln1_g, ln1_b, W_m2, b_m2, ln2_g, ln2_b, W_out, b_out):
    raise NotImplementedError("write your pallas kernel here")


# speedup vs baseline: 2.3416x; 2.3416x over previous
import functools
import jax
import jax.numpy as jnp
from jax import lax
from jax.experimental import pallas as pl
from jax.experimental.pallas import tpu as pltpu
from jax.experimental.pallas import tpu_sc as plsc

F32 = jnp.float32

N = 10000
E = 160000
B = 256
HID = 128
OUT = 128
N_PAD = 10240
NW = 32
CH = 80
NCHUNK = E // CH
ROWS_T = 624
TAIL_T = N - 16 * ROWS_T
SEG_W = B // NW
RD_CH = 64


def _k1_body(x_ref, wn_ref, bn_ref, wp_ref, bp_ref, wa_ref, wb_ref,
             h_ref, p_ref, q_ref):
    h = x_ref[...] @ wn_ref[...] + bn_ref[...]
    h = jnp.maximum(h @ wp_ref[...] + bp_ref[...], 0.0)
    h_ref[...] = h
    p_ref[...] = h @ wa_ref[...]
    q_ref[...] = h @ wb_ref[...]


def _embed_nodes(x, W_nemb, b_nemb, W_proj, b_proj, W1a, W1b):
    blk = 1000
    grid = N // blk
    full = lambda r, c: pl.BlockSpec((r, c), lambda i: (0, 0))
    row = lambda c: pl.BlockSpec((blk, c), lambda i: (i, 0))
    return pl.pallas_call(
        _k1_body,
        grid=(grid,),
        in_specs=[row(128), full(128, 128), full(1, 128), full(128, 128),
                  full(1, 128), full(128, 128), full(128, 128)],
        out_specs=[row(128), row(128), row(128)],
        out_shape=[jax.ShapeDtypeStruct((N, HID), F32)] * 3,
    )(x, W_nemb, b_nemb.reshape(1, -1), W_proj, b_proj.reshape(1, -1),
      W1a, W1b)


def _k2_body(ea_ref, we_ref, be_ref, wc_ref, bm_ref, ce_ref):
    e = ea_ref[...] @ we_ref[...] + be_ref[...]
    ce_ref[...] = e @ wc_ref[...] + bm_ref[...]


def _embed_edges(ea, W_eemb, b_eemb, W1c, b_msg1):
    blk = 2000
    grid = E // blk
    full = lambda r, c: pl.BlockSpec((r, c), lambda i: (0, 0))
    return pl.pallas_call(
        _k2_body,
        grid=(grid,),
        in_specs=[pl.BlockSpec((blk, 16), lambda i: (i, 0)),
                  full(16, 64), full(1, 64), full(64, 128), full(1, 128)],
        out_specs=pl.BlockSpec((blk, 128), lambda i: (i, 0)),
        out_shape=jax.ShapeDtypeStruct((E, HID), F32),
    )(ea, W_eemb, b_eemb.reshape(1, -1), W1c, b_msg1.reshape(1, -1))


def _k3_body(s2_ref, h_ref, degp_ref, w2_ref, b2_ref, wu1_ref, wu2_ref,
             bu_ref, wa_ref, wb_ref, h_out, p_ref, q_ref):
    s = s2_ref[0] + s2_ref[1]
    deg = jnp.sum(degp_ref[...], axis=0)
    agg = s @ w2_ref[...] + deg * b2_ref[...]
    h = jnp.maximum(h_ref[...] @ wu1_ref[...] + agg @ wu2_ref[...]
                    + bu_ref[...], 0.0)
    h_out[...] = h
    p_ref[...] = h @ wa_ref[...]
    q_ref[...] = h @ wb_ref[...]


def _update_nodes(s2, h, degp, W_msg2, b_msg2, Wu1, Wu2, b_upd, W1a, W1b):
    blk = 1000
    grid = N // blk
    full = lambda r, c: pl.BlockSpec((r, c), lambda i: (0, 0))
    row = lambda c: pl.BlockSpec((blk, c), lambda i: (i, 0))
    return pl.pallas_call(
        _k3_body,
        grid=(grid,),
        in_specs=[pl.BlockSpec((2, blk, 128), lambda i: (0, i, 0)),
                  row(128),
                  pl.BlockSpec((2, blk, 1), lambda i: (0, i, 0)),
                  full(128, 128), full(1, 128), full(128, 128),
                  full(128, 128), full(1, 128), full(128, 128),
                  full(128, 128)],
        out_specs=[row(128), row(128), row(128)],
        out_shape=[jax.ShapeDtypeStruct((N, HID), F32)] * 3,
    )(s2, h, degp, W_msg2, b_msg2.reshape(1, -1), Wu1, Wu2,
      b_upd.reshape(1, -1), W1a, W1b)


def _k4_body(h_ref, wn_ref, bn_ref, wa_ref, ba_ref, wn_out, nout_out):
    nout = h_ref[...] @ wn_ref[...] + bn_ref[...]
    w = jax.nn.sigmoid(nout @ wa_ref[...] + ba_ref[...])
    wn_out[...] = w * nout
    nout_out[...] = nout


def _node_out(h_pad, W_nout, b_nout, w_atom, b_atom):
    blk = 1024
    grid = N_PAD // blk
    full = lambda r, c: pl.BlockSpec((r, c), lambda i: (0, 0))
    row = lambda c: pl.BlockSpec((blk, c), lambda i: (i, 0))
    return pl.pallas_call(
        _k4_body,
        grid=(grid,),
        in_specs=[row(128), full(128, 128), full(1, 128), full(128, 1),
                  full(1, 1)],
        out_specs=[row(128), row(128)],
        out_shape=[jax.ShapeDtypeStruct((N_PAD, OUT), F32)] * 2,
    )(h_pad, W_nout, b_nout.reshape(1, -1), w_atom,
      b_atom.reshape(1, 1))


def _ln_in(x, g, b):
    m = jnp.mean(x, axis=-1, keepdims=True)
    v = jnp.var(x, axis=-1, keepdims=True)
    return (x - m) / jnp.sqrt(v + 1e-5) * g + b


def _lrelu(x):
    return jnp.where(x > 0, x, 0.01 * x)


def _k5_body(s0_ref, m0_ref, s1_ref, m1_ref, wb_ref, bb_ref, wm1_ref,
             bm1_ref, g1_ref, c1_ref, wm2_ref, bm2_ref, g2_ref, c2_ref,
             wo_ref, bo_ref, out_ref):
    g0 = jnp.concatenate([s0_ref[...], m0_ref[...]], axis=-1)
    g1 = jnp.concatenate([s1_ref[...], m1_ref[...]], axis=-1)
    t0 = g0 @ wb_ref[...] + bb_ref[...]
    t1 = g1 @ wb_ref[...] + bb_ref[...]
    m = jnp.maximum(t0, t1)
    e0 = jnp.exp(t0 - m)
    e1 = jnp.exp(t1 - m)
    d = e0 + e1
    x = (e0 / d) * g0 + (e1 / d) * g1
    x = _lrelu(_ln_in(x @ wm1_ref[...] + bm1_ref[...], g1_ref[...],
                      c1_ref[...]))
    x = _lrelu(_ln_in(x @ wm2_ref[...] + bm2_ref[...], g2_ref[...],
                      c2_ref[...]))
    out_ref[...] = x @ wo_ref[...] + bo_ref[...]


def _final_mlp(s0, m0, s1, m1, w_beta, b_beta, W_m1, b_m1, ln1_g, ln1_b,
               W_m2, b_m2, ln2_g, ln2_b, W_out, b_out):
    full = lambda r, c: pl.BlockSpec((r, c), lambda: (0, 0))
    return pl.pallas_call(
        _k5_body,
        in_specs=[full(B, 128), full(B, 128), full(B, 128), full(B, 128),
                  full(256, 1), full(1, 1), full(256, 256), full(1, 256),
                  full(1, 256), full(1, 256), full(256, 128), full(1, 128),
                  full(1, 128), full(1, 128), full(128, 1), full(1, 1)],
        out_specs=full(B, 1),
        out_shape=jax.ShapeDtypeStruct((B, 1), F32),
    )(s0, m0, s1, m1, w_beta, b_beta.reshape(1, 1), W_m1,
      b_m1.reshape(1, -1), ln1_g.reshape(1, -1), ln1_b.reshape(1, -1),
      W_m2, b_m2.reshape(1, -1), ln2_g.reshape(1, -1),
      ln2_b.reshape(1, -1), W_out, b_out.reshape(1, 1))


def _make_edge_kernel(with_deg):
    mesh = plsc.VectorSubcoreMesh(core_axis_name="c", subcore_axis_name="s")

    out_type = [jax.ShapeDtypeStruct((2, N, HID), F32)]
    scratch = [
        pltpu.VMEM((CH,), jnp.int32),
        pltpu.VMEM((CH,), jnp.int32),
        pltpu.VMEM((CH, HID), F32),
        pltpu.VMEM((CH, HID), F32),
        pltpu.VMEM_SHARED((N, HID), F32),
        pltpu.SemaphoreType.DMA,
    ]
    if with_deg:
        out_type.append(jax.ShapeDtypeStruct((2 * N,), F32))
        scratch.insert(4, pltpu.VMEM((CH,), F32))
        scratch.insert(5, pltpu.VMEM((ROWS_T,), F32))
        scratch.insert(7, pltpu.VMEM_SHARED((N,), F32))

    def body(p_hbm, q_hbm, ce_hbm, src_hbm, dst_hbm, *refs):
        if with_deg:
            (s_out, deg_out, src_v, dst_v, buf, zbuf, ones_v, deg_stage,
             s_acc, deg_acc, sem) = refs
        else:
            s_out, src_v, dst_v, buf, zbuf, s_acc, sem = refs
        cid = lax.axis_index("c")
        sid = lax.axis_index("s")
        wid = sid * 2 + cid

        zero16 = jnp.zeros((16,), F32)
        for r in range(CH):
            for v in range(8):
                zbuf[r, pl.ds(v * 16, 16)] = zero16

        def zrow(k, _):
            pltpu.sync_copy(zbuf.at[pl.ds(0, 16)],
                            s_acc.at[pl.ds(sid * ROWS_T + k * 16, 16)])
            return 0
        lax.fori_loop(0, ROWS_T // 16, zrow, 0)

        @pl.when(sid == 15)
        def _():
            pltpu.sync_copy(zbuf.at[pl.ds(0, TAIL_T)],
                            s_acc.at[pl.ds(16 * ROWS_T, TAIL_T)])
        if with_deg:
            one16 = jnp.full((16,), 1.0, F32)
            for g in range(CH // 16):
                ones_v[pl.ds(g * 16, 16)] = one16

            def zdeg(k, _):
                pltpu.sync_copy(zbuf.at[0, pl.ds(0, 16)],
                                deg_acc.at[pl.ds(sid * ROWS_T + k * 16, 16)])
                return 0
            lax.fori_loop(0, ROWS_T // 16, zdeg, 0)

            @pl.when(sid == 15)
            def _():
                pltpu.sync_copy(zbuf.at[0, pl.ds(0, TAIL_T)],
                                deg_acc.at[pl.ds(16 * ROWS_T, TAIL_T)])
        plsc.subcore_barrier()

        nchunks_w = (NCHUNK - wid + NW - 1) // NW

        def chunk(k, _):
            base = (wid + k * NW) * CH
            pltpu.sync_copy(src_hbm.at[pl.ds(base, CH)], src_v)
            pltpu.sync_copy(dst_hbm.at[pl.ds(base, CH)], dst_v)
            pltpu.sync_copy(ce_hbm.at[pl.ds(base, CH)], buf)
            pltpu.async_copy(p_hbm.at[src_v], buf, sem, add=True).wait()
            pltpu.async_copy(q_hbm.at[dst_v], buf, sem, add=True).wait()

            def relu_row(r, _):
                for v in range(8):
                    sl = pl.ds(v * 16, 16)
                    buf[r, sl] = jnp.maximum(buf[r, sl], 0.0)
                return 0
            lax.fori_loop(0, CH, relu_row, 0)

            pltpu.sync_copy(buf, s_acc.at[dst_v], add=True)
            if with_deg:
                pltpu.sync_copy(ones_v, deg_acc.at[dst_v], add=True)
            return 0

        lax.fori_loop(0, nchunks_w, chunk, 0)
        plsc.subcore_barrier()

        pltpu.sync_copy(s_acc.at[pl.ds(sid * ROWS_T, ROWS_T)],
                        s_out.at[cid, pl.ds(sid * ROWS_T, ROWS_T)])

        @pl.when(sid == 15)
        def _():
            pltpu.sync_copy(s_acc.at[pl.ds(16 * ROWS_T, TAIL_T)],
                            s_out.at[cid, pl.ds(16 * ROWS_T, TAIL_T)])
        if with_deg:
            pltpu.sync_copy(deg_acc.at[pl.ds(sid * ROWS_T, ROWS_T)],
                            deg_stage)
            pltpu.sync_copy(deg_stage,
                            deg_out.at[pl.ds(cid * N + sid * ROWS_T, ROWS_T)])

            @pl.when(sid == 15)
            def _():
                pltpu.sync_copy(deg_acc.at[pl.ds(16 * ROWS_T, TAIL_T)],
                                deg_stage.at[pl.ds(0, TAIL_T)])
                pltpu.sync_copy(
                    deg_stage.at[pl.ds(0, TAIL_T)],
                    deg_out.at[pl.ds(cid * N + 16 * ROWS_T, TAIL_T)])

    return pl.kernel(body, out_type=tuple(out_type), mesh=mesh,
                     scratch_types=scratch)


_edge_kernel_deg = _make_edge_kernel(True)
_edge_kernel = _make_edge_kernel(False)


def _readout_body(gid_hbm, wn_hbm, nout_hbm, s_out, m_out,
                  gid_v, wn_v, nt_v, acc_s, acc_m, cnt_v, sem):
    cid = lax.axis_index("c")
    sid = lax.axis_index("s")
    wid = sid * 2 + cid
    seg_lo = wid * SEG_W

    pltpu.sync_copy(gid_hbm, gid_v)

    def count_below(bound):
        def step(i, cnt):
            g = gid_v[pl.ds(i * 16, 16)]
            return cnt + jnp.where(g < bound, 1, 0).astype(jnp.int32)
        cv = lax.fori_loop(0, N // 16, step, jnp.zeros((16,), jnp.int32))
        tot = cv[0]
        for l in range(1, 16):
            tot = tot + cv[l]
        return tot

    bounds = [count_below(seg_lo + s) for s in range(SEG_W + 1)]

    for s in range(SEG_W):
        st = bounds[s]
        en = bounds[s + 1]
        st8 = (st // 8) * 8
        nch = (en - st8 + RD_CH - 1) // RD_CH

        def chunk(c, carry):
            pos = st8 + c * RD_CH
            pltpu.sync_copy(wn_hbm.at[pl.ds(pos, RD_CH)], wn_v)
            pltpu.sync_copy(nout_hbm.at[pl.ds(pos, RD_CH)], nt_v)
            j0 = jnp.maximum(0, st - pos)
            j1 = jnp.minimum(RD_CH, en - pos)

            def row(j, acc):
                new = []
                for v in range(8):
                    sl = pl.ds(v * 16, 16)
                    new.append(acc[v] + wn_v[j, sl])
                for v in range(8):
                    sl = pl.ds(v * 16, 16)
                    new.append(jnp.maximum(acc[8 + v], nt_v[j, sl]))
                return tuple(new)

            return lax.fori_loop(j0, j1, row, carry)

        init = tuple([jnp.zeros((16,), F32)] * 8
                     + [jnp.full((16,), -jnp.inf, F32)] * 8)
        acc = lax.fori_loop(0, nch, chunk, init)
        for v in range(8):
            sl = pl.ds(v * 16, 16)
            acc_s[s, sl] = acc[v]
            acc_m[s, sl] = acc[8 + v]

    pltpu.sync_copy(acc_s, s_out.at[pl.ds(seg_lo, SEG_W)])
    pltpu.sync_copy(acc_m, m_out.at[pl.ds(seg_lo, SEG_W)])


_readout_kernel = pl.kernel(
    _readout_body,
    out_type=(jax.ShapeDtypeStruct((B, OUT), F32),
              jax.ShapeDtypeStruct((B, OUT), F32)),
    mesh=plsc.VectorSubcoreMesh(core_axis_name="c", subcore_axis_name="s"),
    scratch_types=[
        pltpu.VMEM((N,), jnp.int32),
        pltpu.VMEM((RD_CH, OUT), F32),
        pltpu.VMEM((RD_CH, OUT), F32),
        pltpu.VMEM((SEG_W, OUT), F32),
        pltpu.VMEM((SEG_W, OUT), F32),
        pltpu.VMEM((16,), jnp.int32),
        pltpu.SemaphoreType.DMA,
    ])


def kernel(x0, x1, edge_attr0, edge_attr1, edge_index0, edge_index1,
           graph_ids0, graph_ids1, W_nemb, b_nemb, W_eemb, b_eemb,
           W_proj, b_proj, W_msg1, b_msg1, W_msg2, b_msg2, W_upd, b_upd,
           W_nout, b_nout, w_atom, b_atom, w_beta, b_beta, W_m1, b_m1,
           ln1_g, ln1_b, W_m2, b_m2, ln2_g, ln2_b, W_out, b_out):
    W1a = W_msg1[:HID]
    W1b = W_msg1[HID:2 * HID]
    W1c = W_msg1[2 * HID:]
    Wu1 = W_upd[:HID]
    Wu2 = W_upd[HID:]

    def conf(x, ea, ei, gid):
        src = ei[0].astype(jnp.int32)
        dst = ei[1].astype(jnp.int32)
        h, P, Q = _embed_nodes(x, W_nemb, b_nemb, W_proj, b_proj, W1a, W1b)
        Ce = _embed_edges(ea, W_eemb, b_eemb, W1c, b_msg1)

        s2, degp = _edge_kernel_deg(P, Q, Ce, src, dst)
        degp = degp.reshape(2, N, 1)
        h, P, Q = _update_nodes(s2, h, degp, W_msg2, b_msg2, Wu1, Wu2,
                                b_upd, W1a, W1b)
        for _ in range(2):
            (s2,) = _edge_kernel(P, Q, Ce, src, dst)
            h, P, Q = _update_nodes(s2, h, degp, W_msg2, b_msg2, Wu1, Wu2,
                                    b_upd, W1a, W1b)

        h_pad = jnp.pad(h, ((0, N_PAD - N), (0, 0)))
        wn, nout = _node_out(h_pad, W_nout, b_nout, w_atom, b_atom)
        s, mx = _readout_kernel(gid.astype(jnp.int32), wn, nout)
        return s, mx

    s0, m0 = conf(x0, edge_attr0, edge_index0, graph_ids0)
    s1, m1 = conf(x1, edge_attr1, edge_index1, graph_ids1)
    out = _final_mlp(s0, m0, s1, m1, w_beta, b_beta, W_m1, b_m1, ln1_g,
                     ln1_b, W_m2, b_m2, ln2_g, ln2_b, W_out, b_out)
    return out[:, 0]


# speedup vs baseline: 3.6930x; 1.5771x over previous
import functools
import jax
import jax.numpy as jnp
from jax import lax
from jax.experimental import pallas as pl
from jax.experimental.pallas import tpu as pltpu
from jax.experimental.pallas import tpu_sc as plsc

F32 = jnp.float32

N = 10000
E = 160000
B = 256
HID = 128
OUT = 128
N_PAD = 10240
NW = 32
CH = 40
PER_W = E // NW
NCH_W = PER_W // CH
ROWS_T = 624
TAIL_T = N - 16 * ROWS_T
SEG_W = B // NW
RD_CH = 64


def _k1_body(x_ref, wn_ref, bn_ref, wp_ref, bp_ref, wa_ref, wb_ref,
             h_ref, p_ref, q_ref):
    h = x_ref[...] @ wn_ref[...] + bn_ref[...]
    h = jnp.maximum(h @ wp_ref[...] + bp_ref[...], 0.0)
    h_ref[...] = h
    p_ref[...] = h @ wa_ref[...]
    q_ref[...] = h @ wb_ref[...]


def _embed_nodes(x, W_nemb, b_nemb, W_proj, b_proj, W1a, W1b):
    blk = 1000
    grid = N // blk
    full = lambda r, c: pl.BlockSpec((r, c), lambda i: (0, 0))
    row = lambda c: pl.BlockSpec((blk, c), lambda i: (i, 0))
    return pl.pallas_call(
        _k1_body,
        grid=(grid,),
        in_specs=[row(128), full(128, 128), full(1, 128), full(128, 128),
                  full(1, 128), full(128, 128), full(128, 128)],
        out_specs=[row(128), row(128), row(128)],
        out_shape=[jax.ShapeDtypeStruct((N, HID), F32)] * 3,
    )(x, W_nemb, b_nemb.reshape(1, -1), W_proj, b_proj.reshape(1, -1),
      W1a, W1b)


def _k2_body(ea_ref, we_ref, be_ref, wc_ref, bm_ref, ce_ref):
    e = ea_ref[...] @ we_ref[...] + be_ref[...]
    ce_ref[...] = e @ wc_ref[...] + bm_ref[...]


def _embed_edges(ea, W_eemb, b_eemb, W1c, b_msg1):
    blk = 2000
    grid = E // blk
    full = lambda r, c: pl.BlockSpec((r, c), lambda i: (0, 0))
    return pl.pallas_call(
        _k2_body,
        grid=(grid,),
        in_specs=[pl.BlockSpec((blk, 16), lambda i: (i, 0)),
                  full(16, 64), full(1, 64), full(64, 128), full(1, 128)],
        out_specs=pl.BlockSpec((blk, 128), lambda i: (i, 0)),
        out_shape=jax.ShapeDtypeStruct((E, HID), F32),
    )(ea, W_eemb, b_eemb.reshape(1, -1), W1c, b_msg1.reshape(1, -1))


def _k3_body(s2_ref, h_ref, degp_ref, w2_ref, b2_ref, wu1_ref, wu2_ref,
             bu_ref, wa_ref, wb_ref, h_out, p_ref, q_ref):
    s = s2_ref[0] + s2_ref[1]
    deg = jnp.sum(degp_ref[...], axis=0)
    agg = s @ w2_ref[...] + deg * b2_ref[...]
    h = jnp.maximum(h_ref[...] @ wu1_ref[...] + agg @ wu2_ref[...]
                    + bu_ref[...], 0.0)
    h_out[...] = h
    p_ref[...] = h @ wa_ref[...]
    q_ref[...] = h @ wb_ref[...]


def _update_nodes(s2, h, degp, W_msg2, b_msg2, Wu1, Wu2, b_upd, W1a, W1b):
    blk = 1000
    grid = N // blk
    full = lambda r, c: pl.BlockSpec((r, c), lambda i: (0, 0))
    row = lambda c: pl.BlockSpec((blk, c), lambda i: (i, 0))
    return pl.pallas_call(
        _k3_body,
        grid=(grid,),
        in_specs=[pl.BlockSpec((2, blk, 128), lambda i: (0, i, 0)),
                  row(128),
                  pl.BlockSpec((2, blk, 1), lambda i: (0, i, 0)),
                  full(128, 128), full(1, 128), full(128, 128),
                  full(128, 128), full(1, 128), full(128, 128),
                  full(128, 128)],
        out_specs=[row(128), row(128), row(128)],
        out_shape=[jax.ShapeDtypeStruct((N, HID), F32)] * 3,
    )(s2, h, degp, W_msg2, b_msg2.reshape(1, -1), Wu1, Wu2,
      b_upd.reshape(1, -1), W1a, W1b)


def _k4_body(h_ref, wn_ref, bn_ref, wa_ref, ba_ref, wn_out, nout_out):
    nout = h_ref[...] @ wn_ref[...] + bn_ref[...]
    w = jax.nn.sigmoid(nout @ wa_ref[...] + ba_ref[...])
    wn_out[...] = w * nout
    nout_out[...] = nout


def _node_out(h_pad, W_nout, b_nout, w_atom, b_atom):
    blk = 1024
    grid = N_PAD // blk
    full = lambda r, c: pl.BlockSpec((r, c), lambda i: (0, 0))
    row = lambda c: pl.BlockSpec((blk, c), lambda i: (i, 0))
    return pl.pallas_call(
        _k4_body,
        grid=(grid,),
        in_specs=[row(128), full(128, 128), full(1, 128), full(128, 1),
                  full(1, 1)],
        out_specs=[row(128), row(128)],
        out_shape=[jax.ShapeDtypeStruct((N_PAD, OUT), F32)] * 2,
    )(h_pad, W_nout, b_nout.reshape(1, -1), w_atom,
      b_atom.reshape(1, 1))


def _ln_in(x, g, b):
    m = jnp.mean(x, axis=-1, keepdims=True)
    v = jnp.var(x, axis=-1, keepdims=True)
    return (x - m) / jnp.sqrt(v + 1e-5) * g + b


def _lrelu(x):
    return jnp.where(x > 0, x, 0.01 * x)


def _k5_body(s0_ref, m0_ref, s1_ref, m1_ref, wb_ref, bb_ref, wm1_ref,
             bm1_ref, g1_ref, c1_ref, wm2_ref, bm2_ref, g2_ref, c2_ref,
             wo_ref, bo_ref, out_ref):
    g0 = jnp.concatenate([s0_ref[...], m0_ref[...]], axis=-1)
    g1 = jnp.concatenate([s1_ref[...], m1_ref[...]], axis=-1)
    t0 = g0 @ wb_ref[...] + bb_ref[...]
    t1 = g1 @ wb_ref[...] + bb_ref[...]
    m = jnp.maximum(t0, t1)
    e0 = jnp.exp(t0 - m)
    e1 = jnp.exp(t1 - m)
    d = e0 + e1
    x = (e0 / d) * g0 + (e1 / d) * g1
    x = _lrelu(_ln_in(x @ wm1_ref[...] + bm1_ref[...], g1_ref[...],
                      c1_ref[...]))
    x = _lrelu(_ln_in(x @ wm2_ref[...] + bm2_ref[...], g2_ref[...],
                      c2_ref[...]))
    out_ref[...] = x @ wo_ref[...] + bo_ref[...]


def _final_mlp(s0, m0, s1, m1, w_beta, b_beta, W_m1, b_m1, ln1_g, ln1_b,
               W_m2, b_m2, ln2_g, ln2_b, W_out, b_out):
    full = lambda r, c: pl.BlockSpec((r, c), lambda: (0, 0))
    return pl.pallas_call(
        _k5_body,
        in_specs=[full(B, 128), full(B, 128), full(B, 128), full(B, 128),
                  full(256, 1), full(1, 1), full(256, 256), full(1, 256),
                  full(1, 256), full(1, 256), full(256, 128), full(1, 128),
                  full(1, 128), full(1, 128), full(128, 1), full(1, 1)],
        out_specs=full(B, 1),
        out_shape=jax.ShapeDtypeStruct((B, 1), F32),
    )(s0, m0, s1, m1, w_beta, b_beta.reshape(1, 1), W_m1,
      b_m1.reshape(1, -1), ln1_g.reshape(1, -1), ln1_b.reshape(1, -1),
      W_m2, b_m2.reshape(1, -1), ln2_g.reshape(1, -1),
      ln2_b.reshape(1, -1), W_out, b_out.reshape(1, 1))


def _make_edge_kernel(with_deg):
    mesh = plsc.VectorSubcoreMesh(core_axis_name="c", subcore_axis_name="s")

    out_type = [jax.ShapeDtypeStruct((2, N, HID), F32)]
    scratch = (
        [pltpu.VMEM((PER_W,), jnp.int32),
         pltpu.VMEM((PER_W,), jnp.int32)]
        + [pltpu.VMEM((CH, HID), F32)] * 6
        + [pltpu.VMEM((CH,), jnp.int32)] * 2
        + [pltpu.VMEM((CH, HID), F32),
           pltpu.VMEM_SHARED((N, HID), F32)]
        + [pltpu.SemaphoreType.DMA] * 4
    )
    if with_deg:
        out_type.append(jax.ShapeDtypeStruct((2 * N,), F32))
        scratch += [
            pltpu.VMEM((CH,), F32),
            pltpu.VMEM((ROWS_T,), F32),
            pltpu.VMEM_SHARED((N,), F32),
            pltpu.SemaphoreType.DMA,
        ]

    def body(p_hbm, q_hbm, ce_hbm, src_hbm, dst_hbm, *refs):
        if with_deg:
            (s_out, deg_out, src_all, dst_all, p0, p1, q0, q1, c0, c1,
             sd0, sd1, zbuf, s_acc, gsem0, gsem1, ssem0, ssem1,
             ones_v, deg_stage, deg_acc, dsem) = refs
        else:
            (s_out, src_all, dst_all, p0, p1, q0, q1, c0, c1,
             sd0, sd1, zbuf, s_acc, gsem0, gsem1, ssem0, ssem1) = refs
        P, Q, C = [p0, p1], [q0, q1], [c0, c1]
        SD, GS, SS = [sd0, sd1], [gsem0, gsem1], [ssem0, ssem1]
        cid = lax.axis_index("c")
        sid = lax.axis_index("s")
        wid = sid * 2 + cid
        wbase = wid * PER_W

        pltpu.sync_copy(src_hbm.at[pl.ds(wbase, PER_W)], src_all)
        pltpu.sync_copy(dst_hbm.at[pl.ds(wbase, PER_W)], dst_all)

        zero16 = jnp.zeros((16,), F32)

        def zfill(r, _):
            for v in range(8):
                zbuf[r, pl.ds(v * 16, 16)] = zero16
            return 0
        lax.fori_loop(0, CH, zfill, 0)

        def zrow(k, _):
            pltpu.sync_copy(zbuf.at[pl.ds(0, 16)],
                            s_acc.at[pl.ds(sid * ROWS_T + k * 16, 16)])
            return 0
        lax.fori_loop(0, ROWS_T // 16, zrow, 0)

        @pl.when(sid == 15)
        def _():
            pltpu.sync_copy(zbuf.at[pl.ds(0, TAIL_T)],
                            s_acc.at[pl.ds(16 * ROWS_T, TAIL_T)])
        if with_deg:
            one16 = jnp.full((16,), 1.0, F32)
            for off in (0, 16, 24):
                ones_v[pl.ds(off, 16)] = one16

            def zdeg(k, _):
                pltpu.sync_copy(zbuf.at[0, pl.ds(0, 16)],
                                deg_acc.at[pl.ds(sid * ROWS_T + k * 16, 16)])
                return 0
            lax.fori_loop(0, ROWS_T // 16, zdeg, 0)

            @pl.when(sid == 15)
            def _():
                pltpu.sync_copy(zbuf.at[0, pl.ds(0, TAIL_T)],
                                deg_acc.at[pl.ds(16 * ROWS_T, TAIL_T)])
        plsc.subcore_barrier()

        def start_gathers(j, b):
            base = wbase + j * CH
            pltpu.async_copy(ce_hbm.at[pl.ds(base, CH)], C[b], GS[b])
            pltpu.async_copy(p_hbm.at[src_all.at[pl.ds(j * CH, CH)]],
                             P[b], GS[b])
            pltpu.async_copy(q_hbm.at[dst_all.at[pl.ds(j * CH, CH)]],
                             Q[b], GS[b])

        def drain_gathers(b):
            for buf in (C[b], P[b], Q[b]):
                pltpu.make_async_copy(ce_hbm.at[pl.ds(wbase, CH)], buf,
                                      GS[b]).wait()

        def drain_scatter(b):
            pltpu.make_async_copy(C[b], s_acc.at[SD[b]], SS[b]).wait()

        def drain_deg():
            pltpu.make_async_copy(ones_v, deg_acc.at[SD[0]], dsem).wait()

        def stage(j, b, start_next, wait_sc, wait_deg):
            drain_gathers(b)
            if wait_sc:
                drain_scatter(1 - b)
            if start_next:
                start_gathers(j + 1, 1 - b)
            if with_deg and wait_deg:
                drain_deg()

            def relu_row(r, _):
                for v in range(8):
                    sl = pl.ds(v * 16, 16)
                    C[b][r, sl] = jnp.maximum(
                        P[b][r, sl] + Q[b][r, sl] + C[b][r, sl], 0.0)
                return 0
            lax.fori_loop(0, CH, relu_row, 0)

            for off in (0, 16, 24):
                SD[b][pl.ds(off, 16)] = dst_all[pl.ds(j * CH + off, 16)]
            pltpu.async_copy(C[b], s_acc.at[SD[b]], SS[b], add=True)
            if with_deg:
                pltpu.async_copy(ones_v, deg_acc.at[SD[b]], dsem, add=True)

        start_gathers(0, 0)
        stage(0, 0, True, False, False)
        stage(1, 1, True, True, True)

        def pair(k, _):
            stage(2 * k, 0, True, True, True)
            stage(2 * k + 1, 1, True, True, True)
            return 0
        lax.fori_loop(1, (NCH_W - 1) // 2, pair, 0)
        stage(NCH_W - 1, 0, False, True, True)
        drain_scatter(0)
        if with_deg:
            drain_deg()
        plsc.subcore_barrier()

        pltpu.sync_copy(s_acc.at[pl.ds(sid * ROWS_T, ROWS_T)],
                        s_out.at[cid, pl.ds(sid * ROWS_T, ROWS_T)])

        @pl.when(sid == 15)
        def _():
            pltpu.sync_copy(s_acc.at[pl.ds(16 * ROWS_T, TAIL_T)],
                            s_out.at[cid, pl.ds(16 * ROWS_T, TAIL_T)])
        if with_deg:
            pltpu.sync_copy(deg_acc.at[pl.ds(sid * ROWS_T, ROWS_T)],
                            deg_stage)
            pltpu.sync_copy(deg_stage,
                            deg_out.at[pl.ds(cid * N + sid * ROWS_T, ROWS_T)])

            @pl.when(sid == 15)
            def _():
                pltpu.sync_copy(deg_acc.at[pl.ds(16 * ROWS_T, TAIL_T)],
                                deg_stage.at[pl.ds(0, TAIL_T)])
                pltpu.sync_copy(
                    deg_stage.at[pl.ds(0, TAIL_T)],
                    deg_out.at[pl.ds(cid * N + 16 * ROWS_T, TAIL_T)])

    return pl.kernel(body, out_type=tuple(out_type), mesh=mesh,
                     scratch_types=scratch)


_edge_kernel_deg = _make_edge_kernel(True)
_edge_kernel = _make_edge_kernel(False)


def _readout_body(gid_hbm, wn_hbm, nout_hbm, s_out, m_out,
                  gid_v, wn_v, nt_v, acc_s, acc_m, cnt_v, sem):
    cid = lax.axis_index("c")
    sid = lax.axis_index("s")
    wid = sid * 2 + cid
    seg_lo = wid * SEG_W

    pltpu.sync_copy(gid_hbm, gid_v)

    def count_below(bound):
        def step(i, cnt):
            g = gid_v[pl.ds(i * 16, 16)]
            return cnt + jnp.where(g < bound, 1, 0).astype(jnp.int32)
        cv = lax.fori_loop(0, N // 16, step, jnp.zeros((16,), jnp.int32))
        tot = cv[0]
        for l in range(1, 16):
            tot = tot + cv[l]
        return tot

    bounds = [count_below(seg_lo + s) for s in range(SEG_W + 1)]

    for s in range(SEG_W):
        st = bounds[s]
        en = bounds[s + 1]
        st8 = (st // 8) * 8
        nch = (en - st8 + RD_CH - 1) // RD_CH

        def chunk(c, carry):
            pos = st8 + c * RD_CH
            pltpu.sync_copy(wn_hbm.at[pl.ds(pos, RD_CH)], wn_v)
            pltpu.sync_copy(nout_hbm.at[pl.ds(pos, RD_CH)], nt_v)
            j0 = jnp.maximum(0, st - pos)
            j1 = jnp.minimum(RD_CH, en - pos)

            def row(j, acc):
                new = []
                for v in range(8):
                    sl = pl.ds(v * 16, 16)
                    new.append(acc[v] + wn_v[j, sl])
                for v in range(8):
                    sl = pl.ds(v * 16, 16)
                    new.append(jnp.maximum(acc[8 + v], nt_v[j, sl]))
                return tuple(new)

            return lax.fori_loop(j0, j1, row, carry)

        init = tuple([jnp.zeros((16,), F32)] * 8
                     + [jnp.full((16,), -jnp.inf, F32)] * 8)
        acc = lax.fori_loop(0, nch, chunk, init)
        for v in range(8):
            sl = pl.ds(v * 16, 16)
            acc_s[s, sl] = acc[v]
            acc_m[s, sl] = acc[8 + v]

    pltpu.sync_copy(acc_s, s_out.at[pl.ds(seg_lo, SEG_W)])
    pltpu.sync_copy(acc_m, m_out.at[pl.ds(seg_lo, SEG_W)])


_readout_kernel = pl.kernel(
    _readout_body,
    out_type=(jax.ShapeDtypeStruct((B, OUT), F32),
              jax.ShapeDtypeStruct((B, OUT), F32)),
    mesh=plsc.VectorSubcoreMesh(core_axis_name="c", subcore_axis_name="s"),
    scratch_types=[
        pltpu.VMEM((N,), jnp.int32),
        pltpu.VMEM((RD_CH, OUT), F32),
        pltpu.VMEM((RD_CH, OUT), F32),
        pltpu.VMEM((SEG_W, OUT), F32),
        pltpu.VMEM((SEG_W, OUT), F32),
        pltpu.VMEM((16,), jnp.int32),
        pltpu.SemaphoreType.DMA,
    ])


def kernel(x0, x1, edge_attr0, edge_attr1, edge_index0, edge_index1,
           graph_ids0, graph_ids1, W_nemb, b_nemb, W_eemb, b_eemb,
           W_proj, b_proj, W_msg1, b_msg1, W_msg2, b_msg2, W_upd, b_upd,
           W_nout, b_nout, w_atom, b_atom, w_beta, b_beta, W_m1, b_m1,
           ln1_g, ln1_b, W_m2, b_m2, ln2_g, ln2_b, W_out, b_out):
    W1a = W_msg1[:HID]
    W1b = W_msg1[HID:2 * HID]
    W1c = W_msg1[2 * HID:]
    Wu1 = W_upd[:HID]
    Wu2 = W_upd[HID:]

    def conf(x, ea, ei, gid):
        src = ei[0].astype(jnp.int32)
        dst = ei[1].astype(jnp.int32)
        h, P, Q = _embed_nodes(x, W_nemb, b_nemb, W_proj, b_proj, W1a, W1b)
        Ce = _embed_edges(ea, W_eemb, b_eemb, W1c, b_msg1)

        s2, degp = _edge_kernel_deg(P, Q, Ce, src, dst)
        degp = degp.reshape(2, N, 1)
        h, P, Q = _update_nodes(s2, h, degp, W_msg2, b_msg2, Wu1, Wu2,
                                b_upd, W1a, W1b)
        for _ in range(2):
            (s2,) = _edge_kernel(P, Q, Ce, src, dst)
            h, P, Q = _update_nodes(s2, h, degp, W_msg2, b_msg2, Wu1, Wu2,
                                    b_upd, W1a, W1b)

        h_pad = jnp.pad(h, ((0, N_PAD - N), (0, 0)))
        wn, nout = _node_out(h_pad, W_nout, b_nout, w_atom, b_atom)
        s, mx = _readout_kernel(gid.astype(jnp.int32), wn, nout)
        return s, mx

    s0, m0 = conf(x0, edge_attr0, edge_index0, graph_ids0)
    s1, m1 = conf(x1, edge_attr1, edge_index1, graph_ids1)
    out = _final_mlp(s0, m0, s1, m1, w_beta, b_beta, W_m1, b_m1, ln1_g,
                     ln1_b, W_m2, b_m2, ln2_g, ln2_b, W_out, b_out)
    return out[:, 0]


# speedup vs baseline: 3.7059x; 1.0035x over previous
import functools
import jax
import jax.numpy as jnp
from jax import lax
from jax.experimental import pallas as pl
from jax.experimental.pallas import tpu as pltpu
from jax.experimental.pallas import tpu_sc as plsc

F32 = jnp.float32

N = 10000
E = 160000
B = 256
HID = 128
OUT = 128
N_PAD = 10240
NW = 32
CH = 40
PER_W = E // NW
NCH_W = PER_W // CH
ROWS_T = 624
TAIL_T = N - 16 * ROWS_T
SEG_W = B // NW
RD_CH = 64


def _k1_body(x_ref, wn_ref, bn_ref, wp_ref, bp_ref, wa_ref, wb_ref,
             h_ref, p_ref, q_ref):
    h = x_ref[...] @ wn_ref[...] + bn_ref[...]
    h = jnp.maximum(h @ wp_ref[...] + bp_ref[...], 0.0)
    h_ref[...] = h
    p_ref[...] = h @ wa_ref[...]
    q_ref[...] = h @ wb_ref[...]


def _embed_nodes(x, W_nemb, b_nemb, W_proj, b_proj, W1a, W1b):
    blk = 1000
    grid = N // blk
    full = lambda r, c: pl.BlockSpec((r, c), lambda i: (0, 0))
    row = lambda c: pl.BlockSpec((blk, c), lambda i: (i, 0))
    return pl.pallas_call(
        _k1_body,
        grid=(grid,),
        in_specs=[row(128), full(128, 128), full(1, 128), full(128, 128),
                  full(1, 128), full(128, 128), full(128, 128)],
        out_specs=[row(128), row(128), row(128)],
        out_shape=[jax.ShapeDtypeStruct((N, HID), F32)] * 3,
    )(x, W_nemb, b_nemb.reshape(1, -1), W_proj, b_proj.reshape(1, -1),
      W1a, W1b)


def _k2_body(ea_ref, we_ref, be_ref, wc_ref, bm_ref, ce_ref):
    e = ea_ref[...] @ we_ref[...] + be_ref[...]
    ce_ref[...] = e @ wc_ref[...] + bm_ref[...]


def _embed_edges(ea, W_eemb, b_eemb, W1c, b_msg1):
    blk = 2000
    grid = E // blk
    full = lambda r, c: pl.BlockSpec((r, c), lambda i: (0, 0))
    return pl.pallas_call(
        _k2_body,
        grid=(grid,),
        in_specs=[pl.BlockSpec((blk, 16), lambda i: (i, 0)),
                  full(16, 64), full(1, 64), full(64, 128), full(1, 128)],
        out_specs=pl.BlockSpec((blk, 128), lambda i: (i, 0)),
        out_shape=jax.ShapeDtypeStruct((E, HID), F32),
    )(ea, W_eemb, b_eemb.reshape(1, -1), W1c, b_msg1.reshape(1, -1))


def _k3_body(s2_ref, h_ref, degp_ref, w2_ref, b2_ref, wu1_ref, wu2_ref,
             bu_ref, wa_ref, wb_ref, h_out, p_ref, q_ref):
    s = s2_ref[0] + s2_ref[1]
    deg = jnp.sum(degp_ref[...], axis=0)
    agg = s @ w2_ref[...] + deg * b2_ref[...]
    h = jnp.maximum(h_ref[...] @ wu1_ref[...] + agg @ wu2_ref[...]
                    + bu_ref[...], 0.0)
    h_out[...] = h
    p_ref[...] = h @ wa_ref[...]
    q_ref[...] = h @ wb_ref[...]


def _update_nodes(s2, h, degp, W_msg2, b_msg2, Wu1, Wu2, b_upd, W1a, W1b):
    blk = 1000
    grid = N // blk
    full = lambda r, c: pl.BlockSpec((r, c), lambda i: (0, 0))
    row = lambda c: pl.BlockSpec((blk, c), lambda i: (i, 0))
    return pl.pallas_call(
        _k3_body,
        grid=(grid,),
        in_specs=[pl.BlockSpec((2, blk, 128), lambda i: (0, i, 0)),
                  row(128),
                  pl.BlockSpec((2, blk, 1), lambda i: (0, i, 0)),
                  full(128, 128), full(1, 128), full(128, 128),
                  full(128, 128), full(1, 128), full(128, 128),
                  full(128, 128)],
        out_specs=[row(128), row(128), row(128)],
        out_shape=[jax.ShapeDtypeStruct((N, HID), F32)] * 3,
    )(s2, h, degp, W_msg2, b_msg2.reshape(1, -1), Wu1, Wu2,
      b_upd.reshape(1, -1), W1a, W1b)


def _k4_body(h_ref, wn_ref, bn_ref, wa_ref, ba_ref, wn_out, nout_out):
    nout = h_ref[...] @ wn_ref[...] + bn_ref[...]
    w = jax.nn.sigmoid(nout @ wa_ref[...] + ba_ref[...])
    wn_out[...] = w * nout
    nout_out[...] = nout


def _node_out(h_pad, W_nout, b_nout, w_atom, b_atom):
    blk = 1024
    grid = N_PAD // blk
    full = lambda r, c: pl.BlockSpec((r, c), lambda i: (0, 0))
    row = lambda c: pl.BlockSpec((blk, c), lambda i: (i, 0))
    return pl.pallas_call(
        _k4_body,
        grid=(grid,),
        in_specs=[row(128), full(128, 128), full(1, 128), full(128, 1),
                  full(1, 1)],
        out_specs=[row(128), row(128)],
        out_shape=[jax.ShapeDtypeStruct((N_PAD, OUT), F32)] * 2,
    )(h_pad, W_nout, b_nout.reshape(1, -1), w_atom,
      b_atom.reshape(1, 1))


def _ln_in(x, g, b):
    m = jnp.mean(x, axis=-1, keepdims=True)
    v = jnp.var(x, axis=-1, keepdims=True)
    return (x - m) / jnp.sqrt(v + 1e-5) * g + b


def _lrelu(x):
    return jnp.where(x > 0, x, 0.01 * x)


def _k5_body(s0_ref, m0_ref, s1_ref, m1_ref, wb_ref, bb_ref, wm1_ref,
             bm1_ref, g1_ref, c1_ref, wm2_ref, bm2_ref, g2_ref, c2_ref,
             wo_ref, bo_ref, out_ref):
    g0 = jnp.concatenate([s0_ref[...], m0_ref[...]], axis=-1)
    g1 = jnp.concatenate([s1_ref[...], m1_ref[...]], axis=-1)
    t0 = g0 @ wb_ref[...] + bb_ref[...]
    t1 = g1 @ wb_ref[...] + bb_ref[...]
    m = jnp.maximum(t0, t1)
    e0 = jnp.exp(t0 - m)
    e1 = jnp.exp(t1 - m)
    d = e0 + e1
    x = (e0 / d) * g0 + (e1 / d) * g1
    x = _lrelu(_ln_in(x @ wm1_ref[...] + bm1_ref[...], g1_ref[...],
                      c1_ref[...]))
    x = _lrelu(_ln_in(x @ wm2_ref[...] + bm2_ref[...], g2_ref[...],
                      c2_ref[...]))
    out_ref[...] = x @ wo_ref[...] + bo_ref[...]


def _final_mlp(s0, m0, s1, m1, w_beta, b_beta, W_m1, b_m1, ln1_g, ln1_b,
               W_m2, b_m2, ln2_g, ln2_b, W_out, b_out):
    full = lambda r, c: pl.BlockSpec((r, c), lambda: (0, 0))
    return pl.pallas_call(
        _k5_body,
        in_specs=[full(B, 128), full(B, 128), full(B, 128), full(B, 128),
                  full(256, 1), full(1, 1), full(256, 256), full(1, 256),
                  full(1, 256), full(1, 256), full(256, 128), full(1, 128),
                  full(1, 128), full(1, 128), full(128, 1), full(1, 1)],
        out_specs=full(B, 1),
        out_shape=jax.ShapeDtypeStruct((B, 1), F32),
    )(s0, m0, s1, m1, w_beta, b_beta.reshape(1, 1), W_m1,
      b_m1.reshape(1, -1), ln1_g.reshape(1, -1), ln1_b.reshape(1, -1),
      W_m2, b_m2.reshape(1, -1), ln2_g.reshape(1, -1),
      ln2_b.reshape(1, -1), W_out, b_out.reshape(1, 1))


def _make_edge_kernel(with_deg):
    mesh = plsc.VectorSubcoreMesh(core_axis_name="c", subcore_axis_name="s")

    out_type = [jax.ShapeDtypeStruct((2, N, HID), F32)]
    scratch = (
        [pltpu.VMEM((PER_W,), jnp.int32),
         pltpu.VMEM((PER_W,), jnp.int32)]
        + [pltpu.VMEM((CH, HID), F32)] * 7
        + [pltpu.VMEM((CH,), jnp.int32)] * 2
        + [pltpu.VMEM((16, HID), F32),
           pltpu.VMEM_SHARED((N, HID), F32)]
        + [pltpu.SemaphoreType.DMA] * 3
    )
    if with_deg:
        out_type.append(jax.ShapeDtypeStruct((2 * N,), F32))
        scratch += [
            pltpu.VMEM((CH,), F32),
            pltpu.VMEM((ROWS_T,), F32),
            pltpu.VMEM_SHARED((N,), F32),
            pltpu.SemaphoreType.DMA,
        ]

    def body(p_hbm, q_hbm, ce_hbm, src_hbm, dst_hbm, *refs):
        if with_deg:
            (s_out, deg_out, src_all, dst_all, p0, p1, q0, q1, c0, c1,
             tbuf, sd0, sd1, zbuf, s_acc, gsem0, gsem1, ssem,
             ones_v, deg_stage, deg_acc, dsem) = refs
        else:
            (s_out, src_all, dst_all, p0, p1, q0, q1, c0, c1,
             tbuf, sd0, sd1, zbuf, s_acc, gsem0, gsem1, ssem) = refs
        P, Q, C = [p0, p1], [q0, q1], [c0, c1]
        SD, GS = [sd0, sd1], [gsem0, gsem1]
        cid = lax.axis_index("c")
        sid = lax.axis_index("s")
        wid = sid * 2 + cid
        wbase = wid * PER_W

        pltpu.sync_copy(src_hbm.at[pl.ds(wbase, PER_W)], src_all)
        pltpu.sync_copy(dst_hbm.at[pl.ds(wbase, PER_W)], dst_all)

        zero16 = jnp.zeros((16,), F32)

        def zfill(r, _):
            for v in range(8):
                zbuf[r, pl.ds(v * 16, 16)] = zero16
            return 0
        lax.fori_loop(0, 16, zfill, 0)

        def zrow(k, _):
            pltpu.async_copy(zbuf,
                             s_acc.at[pl.ds(sid * ROWS_T + k * 16, 16)],
                             gsem0)
            return 0
        lax.fori_loop(0, ROWS_T // 16, zrow, 0)

        def zdrain(k, _):
            pltpu.make_async_copy(
                zbuf, s_acc.at[pl.ds(sid * ROWS_T, 16)], gsem0).wait()
            return 0
        lax.fori_loop(0, ROWS_T // 16, zdrain, 0)

        @pl.when(sid == 15)
        def _():
            pltpu.sync_copy(zbuf.at[pl.ds(0, TAIL_T)],
                            s_acc.at[pl.ds(16 * ROWS_T, TAIL_T)])
        if with_deg:
            one16 = jnp.full((16,), 1.0, F32)
            for off in (0, 16, 24):
                ones_v[pl.ds(off, 16)] = one16

            def zdeg(k, _):
                pltpu.sync_copy(zbuf.at[0, pl.ds(0, 16)],
                                deg_acc.at[pl.ds(sid * ROWS_T + k * 16, 16)])
                return 0
            lax.fori_loop(0, ROWS_T // 16, zdeg, 0)

            @pl.when(sid == 15)
            def _():
                pltpu.sync_copy(zbuf.at[0, pl.ds(0, TAIL_T)],
                                deg_acc.at[pl.ds(16 * ROWS_T, TAIL_T)])
        plsc.subcore_barrier()

        def start_gathers(j, b):
            base = wbase + j * CH
            pltpu.async_copy(ce_hbm.at[pl.ds(base, CH)], C[b], GS[b])
            pltpu.async_copy(p_hbm.at[src_all.at[pl.ds(j * CH, CH)]],
                             P[b], GS[b])
            pltpu.async_copy(q_hbm.at[dst_all.at[pl.ds(j * CH, CH)]],
                             Q[b], GS[b])

        def drain_gathers(b):
            for buf in (C[b], P[b], Q[b]):
                pltpu.make_async_copy(ce_hbm.at[pl.ds(wbase, CH)], buf,
                                      GS[b]).wait()

        def drain_scatter(b):
            pltpu.make_async_copy(tbuf, s_acc.at[SD[b]], ssem).wait()

        def drain_deg():
            pltpu.make_async_copy(ones_v, deg_acc.at[SD[0]], dsem).wait()

        def stage(j, b, start_next, wait_sc, wait_deg):
            drain_gathers(b)
            if start_next:
                start_gathers(j + 1, 1 - b)
            if wait_sc:
                drain_scatter(1 - b)
            if with_deg and wait_deg:
                drain_deg()

            def relu_row(r, _):
                for v in range(8):
                    sl = pl.ds(v * 16, 16)
                    tbuf[r, sl] = jnp.maximum(
                        P[b][r, sl] + Q[b][r, sl] + C[b][r, sl], 0.0)
                return 0
            lax.fori_loop(0, CH, relu_row, 0)

            for off in (0, 16, 24):
                SD[b][pl.ds(off, 16)] = dst_all[pl.ds(j * CH + off, 16)]
            pltpu.async_copy(tbuf, s_acc.at[SD[b]], ssem, add=True)
            if with_deg:
                pltpu.async_copy(ones_v, deg_acc.at[SD[b]], dsem, add=True)

        start_gathers(0, 0)
        stage(0, 0, True, False, False)
        stage(1, 1, True, True, True)

        def pair(k, _):
            stage(2 * k, 0, True, True, True)
            stage(2 * k + 1, 1, True, True, True)
            return 0
        lax.fori_loop(1, (NCH_W - 1) // 2, pair, 0)
        stage(NCH_W - 1, 0, False, True, True)
        drain_scatter(0)
        if with_deg:
            drain_deg()
        plsc.subcore_barrier()

        pltpu.sync_copy(s_acc.at[pl.ds(sid * ROWS_T, ROWS_T)],
                        s_out.at[cid, pl.ds(sid * ROWS_T, ROWS_T)])

        @pl.when(sid == 15)
        def _():
            pltpu.sync_copy(s_acc.at[pl.ds(16 * ROWS_T, TAIL_T)],
                            s_out.at[cid, pl.ds(16 * ROWS_T, TAIL_T)])
        if with_deg:
            pltpu.sync_copy(deg_acc.at[pl.ds(sid * ROWS_T, ROWS_T)],
                            deg_stage)
            pltpu.sync_copy(deg_stage,
                            deg_out.at[pl.ds(cid * N + sid * ROWS_T, ROWS_T)])

            @pl.when(sid == 15)
            def _():
                pltpu.sync_copy(deg_acc.at[pl.ds(16 * ROWS_T, TAIL_T)],
                                deg_stage.at[pl.ds(0, TAIL_T)])
                pltpu.sync_copy(
                    deg_stage.at[pl.ds(0, TAIL_T)],
                    deg_out.at[pl.ds(cid * N + 16 * ROWS_T, TAIL_T)])

    return pl.kernel(body, out_type=tuple(out_type), mesh=mesh,
                     scratch_types=scratch)


_edge_kernel_deg = _make_edge_kernel(True)
_edge_kernel = _make_edge_kernel(False)


def _readout_body(gid_hbm, wn_hbm, nout_hbm, s_out, m_out,
                  gid_v, wn_v, nt_v, acc_s, acc_m, cnt_v, sem):
    cid = lax.axis_index("c")
    sid = lax.axis_index("s")
    wid = sid * 2 + cid
    seg_lo = wid * SEG_W

    pltpu.sync_copy(gid_hbm, gid_v)

    def count_below(bound):
        def step(i, cnt):
            g = gid_v[pl.ds(i * 16, 16)]
            return cnt + jnp.where(g < bound, 1, 0).astype(jnp.int32)
        cv = lax.fori_loop(0, N // 16, step, jnp.zeros((16,), jnp.int32))
        tot = cv[0]
        for l in range(1, 16):
            tot = tot + cv[l]
        return tot

    bounds = [count_below(seg_lo + s) for s in range(SEG_W + 1)]

    for s in range(SEG_W):
        st = bounds[s]
        en = bounds[s + 1]
        st8 = (st // 8) * 8
        nch = (en - st8 + RD_CH - 1) // RD_CH

        def chunk(c, carry):
            pos = st8 + c * RD_CH
            pltpu.sync_copy(wn_hbm.at[pl.ds(pos, RD_CH)], wn_v)
            pltpu.sync_copy(nout_hbm.at[pl.ds(pos, RD_CH)], nt_v)
            j0 = jnp.maximum(0, st - pos)
            j1 = jnp.minimum(RD_CH, en - pos)

            def row(j, acc):
                new = []
                for v in range(8):
                    sl = pl.ds(v * 16, 16)
                    new.append(acc[v] + wn_v[j, sl])
                for v in range(8):
                    sl = pl.ds(v * 16, 16)
                    new.append(jnp.maximum(acc[8 + v], nt_v[j, sl]))
                return tuple(new)

            return lax.fori_loop(j0, j1, row, carry)

        init = tuple([jnp.zeros((16,), F32)] * 8
                     + [jnp.full((16,), -jnp.inf, F32)] * 8)
        acc = lax.fori_loop(0, nch, chunk, init)
        for v in range(8):
            sl = pl.ds(v * 16, 16)
            acc_s[s, sl] = acc[v]
            acc_m[s, sl] = acc[8 + v]

    pltpu.sync_copy(acc_s, s_out.at[pl.ds(seg_lo, SEG_W)])
    pltpu.sync_copy(acc_m, m_out.at[pl.ds(seg_lo, SEG_W)])


_readout_kernel = pl.kernel(
    _readout_body,
    out_type=(jax.ShapeDtypeStruct((B, OUT), F32),
              jax.ShapeDtypeStruct((B, OUT), F32)),
    mesh=plsc.VectorSubcoreMesh(core_axis_name="c", subcore_axis_name="s"),
    scratch_types=[
        pltpu.VMEM((N,), jnp.int32),
        pltpu.VMEM((RD_CH, OUT), F32),
        pltpu.VMEM((RD_CH, OUT), F32),
        pltpu.VMEM((SEG_W, OUT), F32),
        pltpu.VMEM((SEG_W, OUT), F32),
        pltpu.VMEM((16,), jnp.int32),
        pltpu.SemaphoreType.DMA,
    ])


def kernel(x0, x1, edge_attr0, edge_attr1, edge_index0, edge_index1,
           graph_ids0, graph_ids1, W_nemb, b_nemb, W_eemb, b_eemb,
           W_proj, b_proj, W_msg1, b_msg1, W_msg2, b_msg2, W_upd, b_upd,
           W_nout, b_nout, w_atom, b_atom, w_beta, b_beta, W_m1, b_m1,
           ln1_g, ln1_b, W_m2, b_m2, ln2_g, ln2_b, W_out, b_out):
    W1a = W_msg1[:HID]
    W1b = W_msg1[HID:2 * HID]
    W1c = W_msg1[2 * HID:]
    Wu1 = W_upd[:HID]
    Wu2 = W_upd[HID:]

    def conf(x, ea, ei, gid):
        src = ei[0].astype(jnp.int32)
        dst = ei[1].astype(jnp.int32)
        h, P, Q = _embed_nodes(x, W_nemb, b_nemb, W_proj, b_proj, W1a, W1b)
        Ce = _embed_edges(ea, W_eemb, b_eemb, W1c, b_msg1)

        s2, degp = _edge_kernel_deg(P, Q, Ce, src, dst)
        degp = degp.reshape(2, N, 1)
        h, P, Q = _update_nodes(s2, h, degp, W_msg2, b_msg2, Wu1, Wu2,
                                b_upd, W1a, W1b)
        for _ in range(2):
            (s2,) = _edge_kernel(P, Q, Ce, src, dst)
            h, P, Q = _update_nodes(s2, h, degp, W_msg2, b_msg2, Wu1, Wu2,
                                    b_upd, W1a, W1b)

        h_pad = jnp.pad(h, ((0, N_PAD - N), (0, 0)))
        wn, nout = _node_out(h_pad, W_nout, b_nout, w_atom, b_atom)
        s, mx = _readout_kernel(gid.astype(jnp.int32), wn, nout)
        return s, mx

    s0, m0 = conf(x0, edge_attr0, edge_index0, graph_ids0)
    s1, m1 = conf(x1, edge_attr1, edge_index1, graph_ids1)
    out = _final_mlp(s0, m0, s1, m1, w_beta, b_beta, W_m1, b_m1, ln1_g,
                     ln1_b, W_m2, b_m2, ln2_g, ln2_b, W_out, b_out)
    return out[:, 0]


# speedup vs baseline: 3.8397x; 1.0361x over previous
import functools
import jax
import jax.numpy as jnp
from jax import lax
from jax.experimental import pallas as pl
from jax.experimental.pallas import tpu as pltpu
from jax.experimental.pallas import tpu_sc as plsc

F32 = jnp.float32

N = 10000
E = 160000
B = 256
HID = 128
OUT = 128
N_PAD = 10240
NW = 32
CH = 40
PER_W = E // NW
NCH_W = PER_W // CH
ROWS_T = 624
TAIL_T = N - 16 * ROWS_T
SEG_W = B // NW
RD_CH = 64


def _k1_body(x_ref, wn_ref, bn_ref, wp_ref, bp_ref, wa_ref, wb_ref,
             h_ref, p_ref, q_ref):
    h = x_ref[...] @ wn_ref[...] + bn_ref[...]
    h = jnp.maximum(h @ wp_ref[...] + bp_ref[...], 0.0)
    h_ref[...] = h
    p_ref[...] = h @ wa_ref[...]
    q_ref[...] = h @ wb_ref[...]


def _embed_nodes(x, W_nemb, b_nemb, W_proj, b_proj, W1a, W1b):
    blk = 1000
    grid = N // blk
    full = lambda r, c: pl.BlockSpec((r, c), lambda i: (0, 0))
    row = lambda c: pl.BlockSpec((blk, c), lambda i: (i, 0))
    return pl.pallas_call(
        _k1_body,
        grid=(grid,),
        in_specs=[row(128), full(128, 128), full(1, 128), full(128, 128),
                  full(1, 128), full(128, 128), full(128, 128)],
        out_specs=[row(128), row(128), row(128)],
        out_shape=[jax.ShapeDtypeStruct((N, HID), F32)] * 3,
    )(x, W_nemb, b_nemb.reshape(1, -1), W_proj, b_proj.reshape(1, -1),
      W1a, W1b)


def _k2_body(ea_ref, we_ref, be_ref, wc_ref, bm_ref, ce_ref):
    e = ea_ref[...] @ we_ref[...] + be_ref[...]
    ce_ref[...] = e @ wc_ref[...] + bm_ref[...]


def _embed_edges(ea, W_eemb, b_eemb, W1c, b_msg1):
    blk = 2000
    grid = E // blk
    full = lambda r, c: pl.BlockSpec((r, c), lambda i: (0, 0))
    return pl.pallas_call(
        _k2_body,
        grid=(grid,),
        in_specs=[pl.BlockSpec((blk, 16), lambda i: (i, 0)),
                  full(16, 64), full(1, 64), full(64, 128), full(1, 128)],
        out_specs=pl.BlockSpec((blk, 128), lambda i: (i, 0)),
        out_shape=jax.ShapeDtypeStruct((E, HID), F32),
    )(ea, W_eemb, b_eemb.reshape(1, -1), W1c, b_msg1.reshape(1, -1))


def _k3_body(s2_ref, h_ref, degp_ref, w2_ref, b2_ref, wu1_ref, wu2_ref,
             bu_ref, wa_ref, wb_ref, h_out, p_ref, q_ref):
    s = s2_ref[0] + s2_ref[1]
    deg = jnp.sum(degp_ref[...], axis=0)
    agg = s @ w2_ref[...] + deg * b2_ref[...]
    h = jnp.maximum(h_ref[...] @ wu1_ref[...] + agg @ wu2_ref[...]
                    + bu_ref[...], 0.0)
    h_out[...] = h
    p_ref[...] = h @ wa_ref[...]
    q_ref[...] = h @ wb_ref[...]


def _update_nodes(s2, h, degp, W_msg2, b_msg2, Wu1, Wu2, b_upd, W1a, W1b):
    blk = 1000
    grid = N // blk
    full = lambda r, c: pl.BlockSpec((r, c), lambda i: (0, 0))
    row = lambda c: pl.BlockSpec((blk, c), lambda i: (i, 0))
    return pl.pallas_call(
        _k3_body,
        grid=(grid,),
        in_specs=[pl.BlockSpec((2, blk, 128), lambda i: (0, i, 0)),
                  row(128),
                  pl.BlockSpec((2, blk, 1), lambda i: (0, i, 0)),
                  full(128, 128), full(1, 128), full(128, 128),
                  full(128, 128), full(1, 128), full(128, 128),
                  full(128, 128)],
        out_specs=[row(128), row(128), row(128)],
        out_shape=[jax.ShapeDtypeStruct((N, HID), F32)] * 3,
    )(s2, h, degp, W_msg2, b_msg2.reshape(1, -1), Wu1, Wu2,
      b_upd.reshape(1, -1), W1a, W1b)


def _k4_body(h_ref, wn_ref, bn_ref, wa_ref, ba_ref, wn_out, nout_out):
    nout = h_ref[...] @ wn_ref[...] + bn_ref[...]
    w = jax.nn.sigmoid(nout @ wa_ref[...] + ba_ref[...])
    wn_out[...] = w * nout
    nout_out[...] = nout


def _node_out(h_pad, W_nout, b_nout, w_atom, b_atom):
    blk = 1024
    grid = N_PAD // blk
    full = lambda r, c: pl.BlockSpec((r, c), lambda i: (0, 0))
    row = lambda c: pl.BlockSpec((blk, c), lambda i: (i, 0))
    return pl.pallas_call(
        _k4_body,
        grid=(grid,),
        in_specs=[row(128), full(128, 128), full(1, 128), full(128, 1),
                  full(1, 1)],
        out_specs=[row(128), row(128)],
        out_shape=[jax.ShapeDtypeStruct((N_PAD, OUT), F32)] * 2,
    )(h_pad, W_nout, b_nout.reshape(1, -1), w_atom,
      b_atom.reshape(1, 1))


def _ln_in(x, g, b):
    m = jnp.mean(x, axis=-1, keepdims=True)
    v = jnp.var(x, axis=-1, keepdims=True)
    return (x - m) / jnp.sqrt(v + 1e-5) * g + b


def _lrelu(x):
    return jnp.where(x > 0, x, 0.01 * x)


def _k5_body(s0_ref, m0_ref, s1_ref, m1_ref, wb_ref, bb_ref, wm1_ref,
             bm1_ref, g1_ref, c1_ref, wm2_ref, bm2_ref, g2_ref, c2_ref,
             wo_ref, bo_ref, out_ref):
    g0 = jnp.concatenate([s0_ref[...], m0_ref[...]], axis=-1)
    g1 = jnp.concatenate([s1_ref[...], m1_ref[...]], axis=-1)
    t0 = g0 @ wb_ref[...] + bb_ref[...]
    t1 = g1 @ wb_ref[...] + bb_ref[...]
    m = jnp.maximum(t0, t1)
    e0 = jnp.exp(t0 - m)
    e1 = jnp.exp(t1 - m)
    d = e0 + e1
    x = (e0 / d) * g0 + (e1 / d) * g1
    x = _lrelu(_ln_in(x @ wm1_ref[...] + bm1_ref[...], g1_ref[...],
                      c1_ref[...]))
    x = _lrelu(_ln_in(x @ wm2_ref[...] + bm2_ref[...], g2_ref[...],
                      c2_ref[...]))
    out_ref[...] = x @ wo_ref[...] + bo_ref[...]


def _final_mlp(s0, m0, s1, m1, w_beta, b_beta, W_m1, b_m1, ln1_g, ln1_b,
               W_m2, b_m2, ln2_g, ln2_b, W_out, b_out):
    full = lambda r, c: pl.BlockSpec((r, c), lambda: (0, 0))
    return pl.pallas_call(
        _k5_body,
        in_specs=[full(B, 128), full(B, 128), full(B, 128), full(B, 128),
                  full(256, 1), full(1, 1), full(256, 256), full(1, 256),
                  full(1, 256), full(1, 256), full(256, 128), full(1, 128),
                  full(1, 128), full(1, 128), full(128, 1), full(1, 1)],
        out_specs=full(B, 1),
        out_shape=jax.ShapeDtypeStruct((B, 1), F32),
    )(s0, m0, s1, m1, w_beta, b_beta.reshape(1, 1), W_m1,
      b_m1.reshape(1, -1), ln1_g.reshape(1, -1), ln1_b.reshape(1, -1),
      W_m2, b_m2.reshape(1, -1), ln2_g.reshape(1, -1),
      ln2_b.reshape(1, -1), W_out, b_out.reshape(1, 1))


def _make_edge_kernel(with_deg):
    mesh = plsc.VectorSubcoreMesh(core_axis_name="c", subcore_axis_name="s")

    out_type = [jax.ShapeDtypeStruct((2, N, HID), F32)]
    scratch = (
        [pltpu.VMEM((PER_W,), jnp.int32),
         pltpu.VMEM((PER_W,), jnp.int32)]
        + [pltpu.VMEM((CH, HID), F32)] * 4
        + [pltpu.VMEM((CH,), jnp.int32)]
        + [pltpu.VMEM((16, HID), F32),
           pltpu.VMEM_SHARED((N, HID), F32)]
        + [pltpu.SemaphoreType.DMA] * 7
    )
    if with_deg:
        out_type.append(jax.ShapeDtypeStruct((2 * N,), F32))
        scratch += [
            pltpu.VMEM((CH,), F32),
            pltpu.VMEM((ROWS_T,), F32),
            pltpu.VMEM_SHARED((N,), F32),
            pltpu.SemaphoreType.DMA,
        ]

    def body(p_hbm, q_hbm, ce_hbm, src_hbm, dst_hbm, *refs):
        if with_deg:
            (s_out, deg_out, src_all, dst_all, c0, c1, c2, tbuf, sdv,
             zbuf, s_acc, gs0, gs1, gs2, cs0, cs1, cs2, ssem,
             ones_v, deg_stage, deg_acc, dsem) = refs
        else:
            (s_out, src_all, dst_all, c0, c1, c2, tbuf, sdv,
             zbuf, s_acc, gs0, gs1, gs2, cs0, cs1, cs2, ssem) = refs
        C = [c0, c1, c2]
        GS = [gs0, gs1, gs2]
        CS = [cs0, cs1, cs2]
        cid = lax.axis_index("c")
        sid = lax.axis_index("s")
        wid = sid * 2 + cid
        wbase = wid * PER_W

        pltpu.sync_copy(src_hbm.at[pl.ds(wbase, PER_W)], src_all)
        pltpu.sync_copy(dst_hbm.at[pl.ds(wbase, PER_W)], dst_all)

        zero16 = jnp.zeros((16,), F32)

        def zfill(r, _):
            for v in range(8):
                zbuf[r, pl.ds(v * 16, 16)] = zero16
            return 0
        lax.fori_loop(0, 16, zfill, 0)

        def zrow(k, _):
            pltpu.async_copy(zbuf,
                             s_acc.at[pl.ds(sid * ROWS_T + k * 16, 16)],
                             gs0)
            return 0
        lax.fori_loop(0, ROWS_T // 16, zrow, 0)

        def zdrain(k, _):
            pltpu.make_async_copy(
                zbuf, s_acc.at[pl.ds(sid * ROWS_T, 16)], gs0).wait()
            return 0
        lax.fori_loop(0, ROWS_T // 16, zdrain, 0)

        @pl.when(sid == 15)
        def _():
            pltpu.sync_copy(zbuf.at[pl.ds(0, TAIL_T)],
                            s_acc.at[pl.ds(16 * ROWS_T, TAIL_T)])
        if with_deg:
            one16 = jnp.full((16,), 1.0, F32)
            for off in (0, 16, 24):
                ones_v[pl.ds(off, 16)] = one16

            def zdeg(k, _):
                pltpu.sync_copy(zbuf.at[0, pl.ds(0, 16)],
                                deg_acc.at[pl.ds(sid * ROWS_T + k * 16, 16)])
                return 0
            lax.fori_loop(0, ROWS_T // 16, zdeg, 0)

            @pl.when(sid == 15)
            def _():
                pltpu.sync_copy(zbuf.at[0, pl.ds(0, TAIL_T)],
                                deg_acc.at[pl.ds(16 * ROWS_T, TAIL_T)])
        plsc.subcore_barrier()

        def start_ce(j, b):
            pltpu.async_copy(ce_hbm.at[pl.ds(wbase + j * CH, CH)], C[b],
                             CS[b])

        def drain_ce(b):
            pltpu.make_async_copy(ce_hbm.at[pl.ds(wbase, CH)], C[b],
                                  CS[b]).wait()

        def start_pq(j, b):
            pltpu.async_copy(p_hbm.at[src_all.at[pl.ds(j * CH, CH)]],
                             C[b], GS[b], add=True)
            pltpu.async_copy(q_hbm.at[dst_all.at[pl.ds(j * CH, CH)]],
                             C[b], GS[b], add=True)

        def drain_pq(b):
            for _ in range(2):
                pltpu.make_async_copy(ce_hbm.at[pl.ds(wbase, CH)], C[b],
                                      GS[b]).wait()

        def drain_scatter():
            pltpu.make_async_copy(tbuf, s_acc.at[sdv], ssem).wait()

        def drain_deg():
            pltpu.make_async_copy(ones_v, deg_acc.at[sdv], dsem).wait()

        def run_stage(j, b, bn, bn2, start_pq_next, start_ce_next2,
                      wait_sc, wait_deg):
            drain_pq(b)
            if start_pq_next:
                drain_ce(bn)
                start_pq(j + 1, bn)
            if start_ce_next2:
                start_ce(j + 2, bn2)
            if wait_sc:
                drain_scatter()
            if with_deg and wait_deg:
                drain_deg()

            def relu_row(r, _):
                for v in range(8):
                    sl = pl.ds(v * 16, 16)
                    tbuf[r, sl] = jnp.maximum(C[b][r, sl], 0.0)
                return 0
            lax.fori_loop(0, CH, relu_row, 0)

            for off in (0, 16, 24):
                sdv[pl.ds(off, 16)] = dst_all[pl.ds(j * CH + off, 16)]
            pltpu.async_copy(tbuf, s_acc.at[sdv], ssem, add=True)
            if with_deg:
                pltpu.async_copy(ones_v, deg_acc.at[sdv], dsem, add=True)

        start_ce(0, 0)
        drain_ce(0)
        start_pq(0, 0)
        start_ce(1, 1)
        run_stage(0, 0, 1, 2, True, True, False, False)
        run_stage(1, 1, 2, 0, True, True, True, True)

        def triple(k, _):
            j = 3 * k + 2
            run_stage(j, 2, 0, 1, True, True, True, True)
            run_stage(j + 1, 0, 1, 2, True, True, True, True)
            run_stage(j + 2, 1, 2, 0, True, True, True, True)
            return 0
        lax.fori_loop(0, 40, triple, 0)
        run_stage(122, 2, 0, 1, True, True, True, True)
        run_stage(123, 0, 1, 2, True, False, True, True)
        run_stage(124, 1, 2, 0, False, False, True, True)
        drain_scatter()
        if with_deg:
            drain_deg()
        plsc.subcore_barrier()

        plsc.subcore_barrier()

        pltpu.sync_copy(s_acc.at[pl.ds(sid * ROWS_T, ROWS_T)],
                        s_out.at[cid, pl.ds(sid * ROWS_T, ROWS_T)])

        @pl.when(sid == 15)
        def _():
            pltpu.sync_copy(s_acc.at[pl.ds(16 * ROWS_T, TAIL_T)],
                            s_out.at[cid, pl.ds(16 * ROWS_T, TAIL_T)])
        if with_deg:
            pltpu.sync_copy(deg_acc.at[pl.ds(sid * ROWS_T, ROWS_T)],
                            deg_stage)
            pltpu.sync_copy(deg_stage,
                            deg_out.at[pl.ds(cid * N + sid * ROWS_T, ROWS_T)])

            @pl.when(sid == 15)
            def _():
                pltpu.sync_copy(deg_acc.at[pl.ds(16 * ROWS_T, TAIL_T)],
                                deg_stage.at[pl.ds(0, TAIL_T)])
                pltpu.sync_copy(
                    deg_stage.at[pl.ds(0, TAIL_T)],
                    deg_out.at[pl.ds(cid * N + 16 * ROWS_T, TAIL_T)])

    return pl.kernel(body, out_type=tuple(out_type), mesh=mesh,
                     scratch_types=scratch)


_edge_kernel_deg = _make_edge_kernel(True)
_edge_kernel = _make_edge_kernel(False)


def _readout_body(gid_hbm, wn_hbm, nout_hbm, s_out, m_out,
                  gid_v, wn_v, nt_v, acc_s, acc_m, cnt_v, sem):
    cid = lax.axis_index("c")
    sid = lax.axis_index("s")
    wid = sid * 2 + cid
    seg_lo = wid * SEG_W

    pltpu.sync_copy(gid_hbm, gid_v)

    def count_below(bound):
        def step(i, cnt):
            g = gid_v[pl.ds(i * 16, 16)]
            return cnt + jnp.where(g < bound, 1, 0).astype(jnp.int32)
        cv = lax.fori_loop(0, N // 16, step, jnp.zeros((16,), jnp.int32))
        tot = cv[0]
        for l in range(1, 16):
            tot = tot + cv[l]
        return tot

    bounds = [count_below(seg_lo + s) for s in range(SEG_W + 1)]

    for s in range(SEG_W):
        st = bounds[s]
        en = bounds[s + 1]
        st8 = (st // 8) * 8
        nch = (en - st8 + RD_CH - 1) // RD_CH

        def chunk(c, carry):
            pos = st8 + c * RD_CH
            pltpu.sync_copy(wn_hbm.at[pl.ds(pos, RD_CH)], wn_v)
            pltpu.sync_copy(nout_hbm.at[pl.ds(pos, RD_CH)], nt_v)
            j0 = jnp.maximum(0, st - pos)
            j1 = jnp.minimum(RD_CH, en - pos)

            def row(j, acc):
                new = []
                for v in range(8):
                    sl = pl.ds(v * 16, 16)
                    new.append(acc[v] + wn_v[j, sl])
                for v in range(8):
                    sl = pl.ds(v * 16, 16)
                    new.append(jnp.maximum(acc[8 + v], nt_v[j, sl]))
                return tuple(new)

            return lax.fori_loop(j0, j1, row, carry)

        init = tuple([jnp.zeros((16,), F32)] * 8
                     + [jnp.full((16,), -jnp.inf, F32)] * 8)
        acc = lax.fori_loop(0, nch, chunk, init)
        for v in range(8):
            sl = pl.ds(v * 16, 16)
            acc_s[s, sl] = acc[v]
            acc_m[s, sl] = acc[8 + v]

    pltpu.sync_copy(acc_s, s_out.at[pl.ds(seg_lo, SEG_W)])
    pltpu.sync_copy(acc_m, m_out.at[pl.ds(seg_lo, SEG_W)])


_readout_kernel = pl.kernel(
    _readout_body,
    out_type=(jax.ShapeDtypeStruct((B, OUT), F32),
              jax.ShapeDtypeStruct((B, OUT), F32)),
    mesh=plsc.VectorSubcoreMesh(core_axis_name="c", subcore_axis_name="s"),
    scratch_types=[
        pltpu.VMEM((N,), jnp.int32),
        pltpu.VMEM((RD_CH, OUT), F32),
        pltpu.VMEM((RD_CH, OUT), F32),
        pltpu.VMEM((SEG_W, OUT), F32),
        pltpu.VMEM((SEG_W, OUT), F32),
        pltpu.VMEM((16,), jnp.int32),
        pltpu.SemaphoreType.DMA,
    ])


def kernel(x0, x1, edge_attr0, edge_attr1, edge_index0, edge_index1,
           graph_ids0, graph_ids1, W_nemb, b_nemb, W_eemb, b_eemb,
           W_proj, b_proj, W_msg1, b_msg1, W_msg2, b_msg2, W_upd, b_upd,
           W_nout, b_nout, w_atom, b_atom, w_beta, b_beta, W_m1, b_m1,
           ln1_g, ln1_b, W_m2, b_m2, ln2_g, ln2_b, W_out, b_out):
    W1a = W_msg1[:HID]
    W1b = W_msg1[HID:2 * HID]
    W1c = W_msg1[2 * HID:]
    Wu1 = W_upd[:HID]
    Wu2 = W_upd[HID:]

    def conf(x, ea, ei, gid):
        src = ei[0].astype(jnp.int32)
        dst = ei[1].astype(jnp.int32)
        h, P, Q = _embed_nodes(x, W_nemb, b_nemb, W_proj, b_proj, W1a, W1b)
        Ce = _embed_edges(ea, W_eemb, b_eemb, W1c, b_msg1)

        s2, degp = _edge_kernel_deg(P, Q, Ce, src, dst)
        degp = degp.reshape(2, N, 1)
        h, P, Q = _update_nodes(s2, h, degp, W_msg2, b_msg2, Wu1, Wu2,
                                b_upd, W1a, W1b)
        for _ in range(2):
            (s2,) = _edge_kernel(P, Q, Ce, src, dst)
            h, P, Q = _update_nodes(s2, h, degp, W_msg2, b_msg2, Wu1, Wu2,
                                    b_upd, W1a, W1b)

        h_pad = jnp.pad(h, ((0, N_PAD - N), (0, 0)))
        wn, nout = _node_out(h_pad, W_nout, b_nout, w_atom, b_atom)
        s, mx = _readout_kernel(gid.astype(jnp.int32), wn, nout)
        return s, mx

    s0, m0 = conf(x0, edge_attr0, edge_index0, graph_ids0)
    s1, m1 = conf(x1, edge_attr1, edge_index1, graph_ids1)
    out = _final_mlp(s0, m0, s1, m1, w_beta, b_beta, W_m1, b_m1, ln1_g,
                     ln1_b, W_m2, b_m2, ln2_g, ln2_b, W_out, b_out)
    return out[:, 0]


# speedup vs baseline: 3.9427x; 1.0268x over previous
import functools
import jax
import jax.numpy as jnp
from jax import lax
from jax.experimental import pallas as pl
from jax.experimental.pallas import tpu as pltpu
from jax.experimental.pallas import tpu_sc as plsc

F32 = jnp.float32

N = 10000
E = 160000
B = 256
HID = 128
OUT = 128
N_PAD = 10240
NW = 32
CH = 40
PER_W = E // NW
NCH_W = PER_W // CH
ROWS_T = 624
TAIL_T = N - 16 * ROWS_T
SEG_W = B // NW
RD_CH = 64


def _k1_body(x_ref, wn_ref, bn_ref, wp_ref, bp_ref, wa_ref, wb_ref,
             h_ref, p_ref, q_ref):
    h = x_ref[...] @ wn_ref[...] + bn_ref[...]
    h = jnp.maximum(h @ wp_ref[...] + bp_ref[...], 0.0)
    h_ref[...] = h
    p_ref[...] = h @ wa_ref[...]
    q_ref[...] = h @ wb_ref[...]


def _embed_nodes(x, W_nemb, b_nemb, W_proj, b_proj, W1a, W1b):
    blk = 1000
    grid = N // blk
    full = lambda r, c: pl.BlockSpec((r, c), lambda i: (0, 0))
    row = lambda c: pl.BlockSpec((blk, c), lambda i: (i, 0))
    return pl.pallas_call(
        _k1_body,
        grid=(grid,),
        in_specs=[row(128), full(128, 128), full(1, 128), full(128, 128),
                  full(1, 128), full(128, 128), full(128, 128)],
        out_specs=[row(128), row(128), row(128)],
        out_shape=[jax.ShapeDtypeStruct((N, HID), F32)] * 3,
    )(x, W_nemb, b_nemb.reshape(1, -1), W_proj, b_proj.reshape(1, -1),
      W1a, W1b)


def _k2_body(ea_ref, we_ref, be_ref, wc_ref, bm_ref, ce_ref):
    e = ea_ref[...] @ we_ref[...] + be_ref[...]
    ce_ref[...] = e @ wc_ref[...] + bm_ref[...]


def _embed_edges(ea, W_eemb, b_eemb, W1c, b_msg1):
    blk = 2000
    grid = E // blk
    full = lambda r, c: pl.BlockSpec((r, c), lambda i: (0, 0))
    return pl.pallas_call(
        _k2_body,
        grid=(grid,),
        in_specs=[pl.BlockSpec((blk, 16), lambda i: (i, 0)),
                  full(16, 64), full(1, 64), full(64, 128), full(1, 128)],
        out_specs=pl.BlockSpec((blk, 128), lambda i: (i, 0)),
        out_shape=jax.ShapeDtypeStruct((E, HID), F32),
    )(ea, W_eemb, b_eemb.reshape(1, -1), W1c, b_msg1.reshape(1, -1))


def _k3_body(s2_ref, h_ref, degp_ref, w2_ref, b2_ref, wu1_ref, wu2_ref,
             bu_ref, wa_ref, wb_ref, h_out, p_ref, q_ref):
    s = s2_ref[0] + s2_ref[1]
    deg = jnp.sum(degp_ref[...], axis=0)
    agg = s @ w2_ref[...] + deg * b2_ref[...]
    h = jnp.maximum(h_ref[...] @ wu1_ref[...] + agg @ wu2_ref[...]
                    + bu_ref[...], 0.0)
    h_out[...] = h
    p_ref[...] = h @ wa_ref[...]
    q_ref[...] = h @ wb_ref[...]


def _update_nodes(s2, h, degp, W_msg2, b_msg2, Wu1, Wu2, b_upd, W1a, W1b):
    blk = 1000
    grid = N // blk
    full = lambda r, c: pl.BlockSpec((r, c), lambda i: (0, 0))
    row = lambda c: pl.BlockSpec((blk, c), lambda i: (i, 0))
    return pl.pallas_call(
        _k3_body,
        grid=(grid,),
        in_specs=[pl.BlockSpec((2, blk, 128), lambda i: (0, i, 0)),
                  row(128),
                  pl.BlockSpec((2, blk, 1), lambda i: (0, i, 0)),
                  full(128, 128), full(1, 128), full(128, 128),
                  full(128, 128), full(1, 128), full(128, 128),
                  full(128, 128)],
        out_specs=[row(128), row(128), row(128)],
        out_shape=[jax.ShapeDtypeStruct((N, HID), F32)] * 3,
    )(s2, h, degp, W_msg2, b_msg2.reshape(1, -1), Wu1, Wu2,
      b_upd.reshape(1, -1), W1a, W1b)


def _k4_body(h_ref, wn_ref, bn_ref, wa_ref, ba_ref, wn_out, nout_out):
    nout = h_ref[...] @ wn_ref[...] + bn_ref[...]
    w = jax.nn.sigmoid(nout @ wa_ref[...] + ba_ref[...])
    wn_out[...] = w * nout
    nout_out[...] = nout


def _node_out(h_pad, W_nout, b_nout, w_atom, b_atom):
    blk = 1024
    grid = N_PAD // blk
    full = lambda r, c: pl.BlockSpec((r, c), lambda i: (0, 0))
    row = lambda c: pl.BlockSpec((blk, c), lambda i: (i, 0))
    return pl.pallas_call(
        _k4_body,
        grid=(grid,),
        in_specs=[row(128), full(128, 128), full(1, 128), full(128, 1),
                  full(1, 1)],
        out_specs=[row(128), row(128)],
        out_shape=[jax.ShapeDtypeStruct((N_PAD, OUT), F32)] * 2,
    )(h_pad, W_nout, b_nout.reshape(1, -1), w_atom,
      b_atom.reshape(1, 1))


def _ln_in(x, g, b):
    m = jnp.mean(x, axis=-1, keepdims=True)
    v = jnp.var(x, axis=-1, keepdims=True)
    return (x - m) / jnp.sqrt(v + 1e-5) * g + b


def _lrelu(x):
    return jnp.where(x > 0, x, 0.01 * x)


def _k5_body(s0_ref, m0_ref, s1_ref, m1_ref, wb_ref, bb_ref, wm1_ref,
             bm1_ref, g1_ref, c1_ref, wm2_ref, bm2_ref, g2_ref, c2_ref,
             wo_ref, bo_ref, out_ref):
    g0 = jnp.concatenate([s0_ref[...], m0_ref[...]], axis=-1)
    g1 = jnp.concatenate([s1_ref[...], m1_ref[...]], axis=-1)
    t0 = g0 @ wb_ref[...] + bb_ref[...]
    t1 = g1 @ wb_ref[...] + bb_ref[...]
    m = jnp.maximum(t0, t1)
    e0 = jnp.exp(t0 - m)
    e1 = jnp.exp(t1 - m)
    d = e0 + e1
    x = (e0 / d) * g0 + (e1 / d) * g1
    x = _lrelu(_ln_in(x @ wm1_ref[...] + bm1_ref[...], g1_ref[...],
                      c1_ref[...]))
    x = _lrelu(_ln_in(x @ wm2_ref[...] + bm2_ref[...], g2_ref[...],
                      c2_ref[...]))
    out_ref[...] = x @ wo_ref[...] + bo_ref[...]


def _final_mlp(s0, m0, s1, m1, w_beta, b_beta, W_m1, b_m1, ln1_g, ln1_b,
               W_m2, b_m2, ln2_g, ln2_b, W_out, b_out):
    full = lambda r, c: pl.BlockSpec((r, c), lambda: (0, 0))
    return pl.pallas_call(
        _k5_body,
        in_specs=[full(B, 128), full(B, 128), full(B, 128), full(B, 128),
                  full(256, 1), full(1, 1), full(256, 256), full(1, 256),
                  full(1, 256), full(1, 256), full(256, 128), full(1, 128),
                  full(1, 128), full(1, 128), full(128, 1), full(1, 1)],
        out_specs=full(B, 1),
        out_shape=jax.ShapeDtypeStruct((B, 1), F32),
    )(s0, m0, s1, m1, w_beta, b_beta.reshape(1, 1), W_m1,
      b_m1.reshape(1, -1), ln1_g.reshape(1, -1), ln1_b.reshape(1, -1),
      W_m2, b_m2.reshape(1, -1), ln2_g.reshape(1, -1),
      ln2_b.reshape(1, -1), W_out, b_out.reshape(1, 1))


def _make_edge_kernel(with_deg):
    mesh = plsc.VectorSubcoreMesh(core_axis_name="c", subcore_axis_name="s")

    out_type = [jax.ShapeDtypeStruct((2, N, HID), F32)]
    scratch = (
        [pltpu.VMEM((PER_W,), jnp.int32),
         pltpu.VMEM((PER_W,), jnp.int32)]
        + [pltpu.VMEM((CH, HID), F32)] * 4
        + [pltpu.VMEM((CH,), jnp.int32)]
        + [pltpu.VMEM((16, HID), F32),
           pltpu.VMEM_SHARED((N, HID), F32)]
        + [pltpu.SemaphoreType.DMA] * 7
    )
    if with_deg:
        out_type.append(jax.ShapeDtypeStruct((2 * N,), F32))
        scratch += [
            pltpu.VMEM((CH,), F32),
            pltpu.VMEM((ROWS_T,), F32),
            pltpu.VMEM_SHARED((N,), F32),
            pltpu.SemaphoreType.DMA,
        ]

    def body(p_hbm, q_hbm, ce_hbm, src_hbm, dst_hbm, *refs):
        if with_deg:
            (s_out, deg_out, src_all, dst_all, c0, c1, c2, tbuf, sdv,
             zbuf, s_acc, gs0, gs1, gs2, cs0, cs1, cs2, ssem,
             ones_v, deg_stage, deg_acc, dsem) = refs
        else:
            (s_out, src_all, dst_all, c0, c1, c2, tbuf, sdv,
             zbuf, s_acc, gs0, gs1, gs2, cs0, cs1, cs2, ssem) = refs
        C = [c0, c1, c2]
        GS = [gs0, gs1, gs2]
        CS = [cs0, cs1, cs2]
        cid = lax.axis_index("c")
        sid = lax.axis_index("s")
        wid = sid * 2 + cid
        wbase = wid * PER_W

        pltpu.sync_copy(src_hbm.at[pl.ds(wbase, PER_W)], src_all)
        pltpu.sync_copy(dst_hbm.at[pl.ds(wbase, PER_W)], dst_all)

        zero16 = jnp.zeros((16,), F32)

        def zfill(r, _):
            for v in range(8):
                zbuf[r, pl.ds(v * 16, 16)] = zero16
            return 0
        lax.fori_loop(0, 16, zfill, 0)

        def zrow(k, _):
            pltpu.async_copy(zbuf,
                             s_acc.at[pl.ds(sid * ROWS_T + k * 16, 16)],
                             gs0)
            return 0
        lax.fori_loop(0, ROWS_T // 16, zrow, 0)

        def zdrain(k, _):
            pltpu.make_async_copy(
                zbuf, s_acc.at[pl.ds(sid * ROWS_T, 16)], gs0).wait()
            return 0
        lax.fori_loop(0, ROWS_T // 16, zdrain, 0)

        @pl.when(sid == 15)
        def _():
            pltpu.sync_copy(zbuf.at[pl.ds(0, TAIL_T)],
                            s_acc.at[pl.ds(16 * ROWS_T, TAIL_T)])
        if with_deg:
            one16 = jnp.full((16,), 1.0, F32)
            for off in (0, 16, 24):
                ones_v[pl.ds(off, 16)] = one16

            def zdeg(k, _):
                pltpu.sync_copy(zbuf.at[0, pl.ds(0, 16)],
                                deg_acc.at[pl.ds(sid * ROWS_T + k * 16, 16)])
                return 0
            lax.fori_loop(0, ROWS_T // 16, zdeg, 0)

            @pl.when(sid == 15)
            def _():
                pltpu.sync_copy(zbuf.at[0, pl.ds(0, TAIL_T)],
                                deg_acc.at[pl.ds(16 * ROWS_T, TAIL_T)])
        plsc.subcore_barrier()

        def start_ce(j, b):
            pltpu.async_copy(ce_hbm.at[pl.ds(wbase + j * CH, CH)], C[b],
                             CS[b])

        def drain_ce(b):
            pltpu.make_async_copy(ce_hbm.at[pl.ds(wbase, CH)], C[b],
                                  CS[b]).wait()

        def start_pq(j, b):
            pltpu.async_copy(p_hbm.at[src_all.at[pl.ds(j * CH, CH)]],
                             C[b], GS[b], add=True)
            pltpu.async_copy(q_hbm.at[dst_all.at[pl.ds(j * CH, CH)]],
                             C[b], GS[b], add=True)

        def drain_pq(b):
            for _ in range(2):
                pltpu.make_async_copy(ce_hbm.at[pl.ds(wbase, CH)], C[b],
                                      GS[b]).wait()

        def drain_scatter():
            pltpu.make_async_copy(tbuf, s_acc.at[sdv], ssem).wait()

        def drain_deg():
            pltpu.make_async_copy(ones_v, deg_acc.at[sdv], dsem).wait()

        def run_stage(j, b, bn, bn2, start_pq_next, start_ce_next2,
                      wait_sc, wait_deg):
            drain_pq(b)
            if start_pq_next:
                drain_ce(bn)
                start_pq(j + 1, bn)
            if start_ce_next2:
                start_ce(j + 2, bn2)
            if wait_sc:
                drain_scatter()
            if with_deg and wait_deg:
                drain_deg()

            def relu_row(r, _):
                for v in range(8):
                    sl = pl.ds(v * 16, 16)
                    tbuf[r, sl] = jnp.maximum(C[b][r, sl], 0.0)
                return 0
            lax.fori_loop(0, CH, relu_row, 0)

            for off in (0, 16, 24):
                sdv[pl.ds(off, 16)] = dst_all[pl.ds(j * CH + off, 16)]
            pltpu.async_copy(tbuf, s_acc.at[sdv], ssem, add=True)
            if with_deg:
                pltpu.async_copy(ones_v, deg_acc.at[sdv], dsem, add=True)

        start_ce(0, 0)
        drain_ce(0)
        start_pq(0, 0)
        start_ce(1, 1)
        run_stage(0, 0, 1, 2, True, True, False, False)
        run_stage(1, 1, 2, 0, True, True, True, True)

        def triple(k, _):
            j = 3 * k + 2
            run_stage(j, 2, 0, 1, True, True, True, True)
            run_stage(j + 1, 0, 1, 2, True, True, True, True)
            run_stage(j + 2, 1, 2, 0, True, True, True, True)
            return 0
        lax.fori_loop(0, 40, triple, 0)
        run_stage(122, 2, 0, 1, True, True, True, True)
        run_stage(123, 0, 1, 2, True, False, True, True)
        run_stage(124, 1, 2, 0, False, False, True, True)
        drain_scatter()
        if with_deg:
            drain_deg()
        plsc.subcore_barrier()

        plsc.subcore_barrier()

        pltpu.sync_copy(s_acc.at[pl.ds(sid * ROWS_T, ROWS_T)],
                        s_out.at[cid, pl.ds(sid * ROWS_T, ROWS_T)])

        @pl.when(sid == 15)
        def _():
            pltpu.sync_copy(s_acc.at[pl.ds(16 * ROWS_T, TAIL_T)],
                            s_out.at[cid, pl.ds(16 * ROWS_T, TAIL_T)])
        if with_deg:
            pltpu.sync_copy(deg_acc.at[pl.ds(sid * ROWS_T, ROWS_T)],
                            deg_stage)
            pltpu.sync_copy(deg_stage,
                            deg_out.at[pl.ds(cid * N + sid * ROWS_T, ROWS_T)])

            @pl.when(sid == 15)
            def _():
                pltpu.sync_copy(deg_acc.at[pl.ds(16 * ROWS_T, TAIL_T)],
                                deg_stage.at[pl.ds(0, TAIL_T)])
                pltpu.sync_copy(
                    deg_stage.at[pl.ds(0, TAIL_T)],
                    deg_out.at[pl.ds(cid * N + 16 * ROWS_T, TAIL_T)])

    return pl.kernel(body, out_type=tuple(out_type), mesh=mesh,
                     scratch_types=scratch)


_edge_kernel_deg = _make_edge_kernel(True)
_edge_kernel = _make_edge_kernel(False)


def _readout_body(gid_hbm, wn_hbm, nout_hbm, s_out, m_out,
                  gid_v, wn_v, nt_v, acc_s, acc_m, cnt_v, sem):
    cid = lax.axis_index("c")
    sid = lax.axis_index("s")
    wid = sid * 2 + cid
    seg_lo = wid * SEG_W

    pltpu.sync_copy(gid_hbm, gid_v)

    def count_step(i, cnts):
        g = gid_v[pl.ds(i * 16, 16)]
        return tuple(
            cnts[s] + jnp.where(g < seg_lo + s, 1, 0).astype(jnp.int32)
            for s in range(SEG_W + 1))

    cvs = lax.fori_loop(0, N // 16, count_step,
                        tuple([jnp.zeros((16,), jnp.int32)] * (SEG_W + 1)))

    def lanesum(cv):
        tot = cv[0]
        for l in range(1, 16):
            tot = tot + cv[l]
        return tot

    bounds = [lanesum(cv) for cv in cvs]

    for s in range(SEG_W):
        st = bounds[s]
        en = bounds[s + 1]
        st8 = (st // 8) * 8
        nch = (en - st8 + RD_CH - 1) // RD_CH

        def chunk(c, carry):
            pos = st8 + c * RD_CH
            a = pltpu.async_copy(wn_hbm.at[pl.ds(pos, RD_CH)], wn_v, sem)
            b = pltpu.async_copy(nout_hbm.at[pl.ds(pos, RD_CH)], nt_v, sem)
            a.wait()
            b.wait()
            j0 = jnp.maximum(0, st - pos)
            j1 = jnp.minimum(RD_CH, en - pos)

            def row(j, acc):
                new = []
                for v in range(8):
                    sl = pl.ds(v * 16, 16)
                    new.append(acc[v] + wn_v[j, sl])
                for v in range(8):
                    sl = pl.ds(v * 16, 16)
                    new.append(jnp.maximum(acc[8 + v], nt_v[j, sl]))
                return tuple(new)

            return lax.fori_loop(j0, j1, row, carry)

        init = tuple([jnp.zeros((16,), F32)] * 8
                     + [jnp.full((16,), -jnp.inf, F32)] * 8)
        acc = lax.fori_loop(0, nch, chunk, init)
        for v in range(8):
            sl = pl.ds(v * 16, 16)
            acc_s[s, sl] = acc[v]
            acc_m[s, sl] = acc[8 + v]

    pltpu.sync_copy(acc_s, s_out.at[pl.ds(seg_lo, SEG_W)])
    pltpu.sync_copy(acc_m, m_out.at[pl.ds(seg_lo, SEG_W)])


_readout_kernel = pl.kernel(
    _readout_body,
    out_type=(jax.ShapeDtypeStruct((B, OUT), F32),
              jax.ShapeDtypeStruct((B, OUT), F32)),
    mesh=plsc.VectorSubcoreMesh(core_axis_name="c", subcore_axis_name="s"),
    scratch_types=[
        pltpu.VMEM((N,), jnp.int32),
        pltpu.VMEM((RD_CH, OUT), F32),
        pltpu.VMEM((RD_CH, OUT), F32),
        pltpu.VMEM((SEG_W, OUT), F32),
        pltpu.VMEM((SEG_W, OUT), F32),
        pltpu.VMEM((16,), jnp.int32),
        pltpu.SemaphoreType.DMA,
    ])


def kernel(x0, x1, edge_attr0, edge_attr1, edge_index0, edge_index1,
           graph_ids0, graph_ids1, W_nemb, b_nemb, W_eemb, b_eemb,
           W_proj, b_proj, W_msg1, b_msg1, W_msg2, b_msg2, W_upd, b_upd,
           W_nout, b_nout, w_atom, b_atom, w_beta, b_beta, W_m1, b_m1,
           ln1_g, ln1_b, W_m2, b_m2, ln2_g, ln2_b, W_out, b_out):
    W1a = W_msg1[:HID]
    W1b = W_msg1[HID:2 * HID]
    W1c = W_msg1[2 * HID:]
    Wu1 = W_upd[:HID]
    Wu2 = W_upd[HID:]

    def conf(x, ea, ei, gid):
        src = ei[0].astype(jnp.int32)
        dst = ei[1].astype(jnp.int32)
        h, P, Q = _embed_nodes(x, W_nemb, b_nemb, W_proj, b_proj, W1a, W1b)
        Ce = _embed_edges(ea, W_eemb, b_eemb, W1c, b_msg1)

        s2, degp = _edge_kernel_deg(P, Q, Ce, src, dst)
        degp = degp.reshape(2, N, 1)
        h, P, Q = _update_nodes(s2, h, degp, W_msg2, b_msg2, Wu1, Wu2,
                                b_upd, W1a, W1b)
        for _ in range(2):
            (s2,) = _edge_kernel(P, Q, Ce, src, dst)
            h, P, Q = _update_nodes(s2, h, degp, W_msg2, b_msg2, Wu1, Wu2,
                                    b_upd, W1a, W1b)

        h_pad = jnp.pad(h, ((0, N_PAD - N), (0, 0)))
        wn, nout = _node_out(h_pad, W_nout, b_nout, w_atom, b_atom)
        s, mx = _readout_kernel(gid.astype(jnp.int32), wn, nout)
        return s, mx

    s0, m0 = conf(x0, edge_attr0, edge_index0, graph_ids0)
    s1, m1 = conf(x1, edge_attr1, edge_index1, graph_ids1)
    out = _final_mlp(s0, m0, s1, m1, w_beta, b_beta, W_m1, b_m1, ln1_g,
                     ln1_b, W_m2, b_m2, ln2_g, ln2_b, W_out, b_out)
    return out[:, 0]
